# Initial kernel scaffold; baseline (speedup 1.0000x reference)
import jax
import jax.numpy as jnp
from jax.experimental import pallas as pl


def kernel(n_id, x0, emb1, edge_index1, e_id1, edge_index2, e_id2, edge_type, node_type, local_node_idx, rel_W1, root_W1, root_b1, rel_W2, root_W2, root_b2):
    raise NotImplementedError("write your pallas kernel here")


# speedup vs baseline: 9.9132x; 9.9132x over previous
import functools

import jax
import jax.numpy as jnp
from jax import lax
from jax.experimental import pallas as pl
from jax.experimental.pallas import tpu as pltpu
from jax.experimental.pallas import tpu_sc as plsc

F32 = jnp.float32
I32 = jnp.int32

IN_C = 128
NTY = 2
ETY = 4
N0 = 10000
S1 = 5000
S2 = 2000
E1 = 320000
E2 = 160000
ETOT = 480000
NX0 = 6000

NC = 2
NS = 16
L = 16
K = 80
H = 64
HROWS = 5120
AP1 = 20480
AP2 = 8192


def _sc_layer1(tbl_lo, tbl_hi, nt_h, li_h, src_h, dst_h, eid_h, et_h,
               z64, z8, ones_in,
               h_lo, h_hi, acc_a, acc_b, cnt_o,
               acc_s, cnt_s,
               srcv, dstv, eidv, keyv, etv, rowsv, onesv, ntv, liv, cidxv,
               sem):
    rpt = AP1 // NS
    ept = E1 // NS
    nch = ept // K
    hpt = HROWS // NS

    c = lax.axis_index("c")
    s = lax.axis_index("s")
    is0 = c == 0

    ab = s * rpt
    pltpu.sync_copy(z64, acc_s.at[pl.ds(ab, rpt)])

    @pl.when(is0)
    def _():
        pltpu.sync_copy(z8, cnt_s.at[pl.ds(ab, rpt)])

    pltpu.sync_copy(ones_in, onesv)

    for i in range(hpt // K):
        base = s * hpt + i * K
        pltpu.sync_copy(nt_h.at[pl.ds(base, K)], ntv)
        pltpu.sync_copy(li_h.at[pl.ds(base, K)], liv)
        for j in range(K // L):
            nt16 = ntv[pl.ds(j * L, L)]
            li16 = liv[pl.ds(j * L, L)]
            cidxv[pl.ds(j * L, L)] = jnp.where(nt16 == 0, li16, li16 + NX0)

        @pl.when(is0)
        def _():
            pltpu.async_copy(tbl_lo.at[cidxv], rowsv, sem).wait()
            pltpu.sync_copy(rowsv, h_lo.at[pl.ds(base, K)])

        @pl.when(~is0)
        def _():
            pltpu.async_copy(tbl_hi.at[cidxv], rowsv, sem).wait()
            pltpu.sync_copy(rowsv, h_hi.at[pl.ds(base, K)])

    plsc.subcore_barrier()

    def edge_chunk(i, carry):
        base = s * ept + i * K
        pltpu.sync_copy(src_h.at[pl.ds(base, K)], srcv)
        pltpu.sync_copy(dst_h.at[pl.ds(base, K)], dstv)
        pltpu.sync_copy(eid_h.at[pl.ds(base, K)], eidv)
        pltpu.async_copy(et_h.at[eidv], etv, sem).wait()
        for j in range(K // L):
            et16 = etv[pl.ds(j * L, L)]
            d16 = dstv[pl.ds(j * L, L)]
            keyv[pl.ds(j * L, L)] = et16 * S1 + d16

        @pl.when(is0)
        def _():
            pltpu.async_copy(h_lo.at[srcv], rowsv, sem).wait()

        @pl.when(~is0)
        def _():
            pltpu.async_copy(h_hi.at[srcv], rowsv, sem).wait()

        pltpu.sync_copy(rowsv, acc_s.at[keyv], add=True)

        @pl.when(is0)
        def _():
            pltpu.sync_copy(onesv, cnt_s.at[keyv], add=True)

        return carry

    lax.fori_loop(0, nch, edge_chunk, 0)
    plsc.subcore_barrier()

    @pl.when(is0)
    def _():
        pltpu.sync_copy(acc_s.at[pl.ds(ab, rpt)], acc_a.at[pl.ds(ab, rpt)])
        pltpu.sync_copy(cnt_s.at[pl.ds(ab, rpt)], cnt_o.at[pl.ds(ab, rpt)])

    @pl.when(~is0)
    def _():
        pltpu.sync_copy(acc_s.at[pl.ds(ab, rpt)], acc_b.at[pl.ds(ab, rpt)])


def _sc_layer2(x_lo, x_hi, src_h, dst_h, eid_h, et_h, z64, z8, ones_in,
               acc_a, acc_b, cnt_o,
               acc_s, cnt_s,
               srcv, dstv, eidv, keyv, etv, rowsv, onesv,
               sem):
    rpt = AP2 // NS
    ept = E2 // NS
    nch = ept // K

    c = lax.axis_index("c")
    s = lax.axis_index("s")
    is0 = c == 0

    ab = s * rpt
    pltpu.sync_copy(z64, acc_s.at[pl.ds(ab, rpt)])

    @pl.when(is0)
    def _():
        pltpu.sync_copy(z8, cnt_s.at[pl.ds(ab, rpt)])

    pltpu.sync_copy(ones_in, onesv)
    plsc.subcore_barrier()

    def edge_chunk(i, carry):
        base = s * ept + i * K
        pltpu.sync_copy(src_h.at[pl.ds(base, K)], srcv)
        pltpu.sync_copy(dst_h.at[pl.ds(base, K)], dstv)
        pltpu.sync_copy(eid_h.at[pl.ds(base, K)], eidv)
        pltpu.async_copy(et_h.at[eidv], etv, sem).wait()
        for j in range(K // L):
            et16 = etv[pl.ds(j * L, L)]
            d16 = dstv[pl.ds(j * L, L)]
            keyv[pl.ds(j * L, L)] = et16 * S2 + d16

        @pl.when(is0)
        def _():
            pltpu.async_copy(x_lo.at[srcv], rowsv, sem).wait()

        @pl.when(~is0)
        def _():
            pltpu.async_copy(x_hi.at[srcv], rowsv, sem).wait()

        pltpu.sync_copy(rowsv, acc_s.at[keyv], add=True)

        @pl.when(is0)
        def _():
            pltpu.sync_copy(onesv, cnt_s.at[keyv], add=True)

        return carry

    lax.fori_loop(0, nch, edge_chunk, 0)
    plsc.subcore_barrier()

    @pl.when(is0)
    def _():
        pltpu.sync_copy(acc_s.at[pl.ds(ab, rpt)], acc_a.at[pl.ds(ab, rpt)])
        pltpu.sync_copy(cnt_s.at[pl.ds(ab, rpt)], cnt_o.at[pl.ds(ab, rpt)])

    @pl.when(~is0)
    def _():
        pltpu.sync_copy(acc_s.at[pl.ds(ab, rpt)], acc_b.at[pl.ds(ab, rpt)])


def _make_l1():
    A = AP1
    mesh = plsc.VectorSubcoreMesh(core_axis_name="c", subcore_axis_name="s")
    return pl.kernel(
        _sc_layer1,
        compiler_params=pltpu.CompilerParams(use_tc_tiling_on_sc=False),
        out_type=[
            jax.ShapeDtypeStruct((HROWS, H), F32),
            jax.ShapeDtypeStruct((HROWS, H), F32),
            jax.ShapeDtypeStruct((A, H), F32),
            jax.ShapeDtypeStruct((A, H), F32),
            jax.ShapeDtypeStruct((A, 8), F32),
        ],
        mesh=mesh,
        scratch_types=[
            pltpu.VMEM_SHARED((A, H), F32),
            pltpu.VMEM_SHARED((A, 8), F32),
            pltpu.VMEM((K,), I32),
            pltpu.VMEM((K,), I32),
            pltpu.VMEM((K,), I32),
            pltpu.VMEM((K,), I32),
            pltpu.VMEM((K,), I32),
            pltpu.VMEM((K, H), F32),
            pltpu.VMEM((K, 8), F32),
            pltpu.VMEM((K,), I32),
            pltpu.VMEM((K,), I32),
            pltpu.VMEM((K,), I32),
            pltpu.SemaphoreType.DMA,
        ],
    )


def _make_l2():
    A = AP2
    mesh = plsc.VectorSubcoreMesh(core_axis_name="c", subcore_axis_name="s")
    return pl.kernel(
        _sc_layer2,
        compiler_params=pltpu.CompilerParams(use_tc_tiling_on_sc=False),
        out_type=[
            jax.ShapeDtypeStruct((A, H), F32),
            jax.ShapeDtypeStruct((A, H), F32),
            jax.ShapeDtypeStruct((A, 8), F32),
        ],
        mesh=mesh,
        scratch_types=[
            pltpu.VMEM_SHARED((A, H), F32),
            pltpu.VMEM_SHARED((A, 8), F32),
            pltpu.VMEM((K,), I32),
            pltpu.VMEM((K,), I32),
            pltpu.VMEM((K,), I32),
            pltpu.VMEM((K,), I32),
            pltpu.VMEM((K,), I32),
            pltpu.VMEM((K, H), F32),
            pltpu.VMEM((K, 8), F32),
            pltpu.SemaphoreType.DMA,
        ],
    )


def _dense_body(acc_ref, cnt_ref, x_ref, ntf_ref, w_ref, r_ref, b_ref, o_ref,
                *, last_t, relu, logsm):
    t = pl.program_id(1)

    @pl.when(t == 0)
    def _():
        x = x_ref[...]
        m0 = ntf_ref[...] == 0.0
        r0 = jnp.dot(x, r_ref[0], preferred_element_type=F32) + b_ref[0, :]
        r1 = jnp.dot(x, r_ref[1], preferred_element_type=F32) + b_ref[1, :]
        o_ref[...] = jnp.where(m0, r0, r1)

    inv = 1.0 / jnp.maximum(cnt_ref[0], 1.0)
    a = acc_ref[0] * inv
    o_ref[...] += jnp.dot(a, w_ref[0], preferred_element_type=F32)

    @pl.when(t == last_t)
    def _():
        y = o_ref[...]
        if relu:
            o_ref[...] = jnp.maximum(y, 0.0)
        if logsm:
            m = jnp.max(y, axis=-1, keepdims=True)
            e = jnp.exp(y - m)
            o_ref[...] = y - m - jnp.log(jnp.sum(e, axis=-1, keepdims=True))


def _make_dense(S, blk, relu, logsm):
    rb = S // blk
    body = functools.partial(_dense_body, last_t=ETY - 1, relu=relu,
                             logsm=logsm)
    return pl.pallas_call(
        body,
        grid=(rb, ETY),
        in_specs=[
            pl.BlockSpec((1, blk, IN_C), lambda i, t: (t, i, 0)),
            pl.BlockSpec((1, blk, 1), lambda i, t: (t, i, 0)),
            pl.BlockSpec((blk, IN_C), lambda i, t: (i, 0)),
            pl.BlockSpec((blk, 1), lambda i, t: (i, 0)),
            pl.BlockSpec((1, IN_C, IN_C), lambda i, t: (t, 0, 0)),
            pl.BlockSpec((NTY, IN_C, IN_C), lambda i, t: (0, 0, 0)),
            pl.BlockSpec((8, IN_C), lambda i, t: (0, 0)),
        ],
        out_specs=pl.BlockSpec((blk, IN_C), lambda i, t: (i, 0)),
        out_shape=jax.ShapeDtypeStruct((S, IN_C), F32),
    )


_L1 = _make_l1()
_L2 = _make_l2()
_D1 = _make_dense(S1, 1000, relu=True, logsm=False)
_D2 = _make_dense(S2, 1000, relu=False, logsm=True)


def kernel(n_id, x0, emb1, edge_index1, e_id1, edge_index2, e_id2, edge_type,
           node_type, local_node_idx, rel_W1, root_W1, root_b1, rel_W2,
           root_W2, root_b2):
    tbl = jnp.concatenate([x0, emb1], axis=0)
    tbl_lo = tbl[:, :H]
    tbl_hi = tbl[:, H:]
    z64 = jnp.zeros((AP1 // NS, H), F32)
    z8 = jnp.zeros((AP1 // NS, 8), F32)
    z64b = jnp.zeros((AP2 // NS, H), F32)
    z8b = jnp.zeros((AP2 // NS, 8), F32)
    ones_in = jnp.ones((K, 8), F32)

    h_lo, h_hi, acc1a, acc1b, cnt1 = _L1(
        tbl_lo, tbl_hi, node_type, local_node_idx,
        edge_index1[0], edge_index1[1], e_id1, edge_type,
        z64, z8, ones_in)

    acc1 = jnp.concatenate([acc1a[:ETY * S1].reshape(ETY, S1, H),
                            acc1b[:ETY * S1].reshape(ETY, S1, H)], axis=-1)
    cnt1r = cnt1[:ETY * S1, :1].reshape(ETY, S1, 1)
    hx = jnp.concatenate([h_lo[:S1], h_hi[:S1]], axis=-1)
    ntf1 = node_type[:S1].astype(F32)[:, None]
    b1p = jnp.zeros((8, IN_C), F32).at[:NTY].set(root_b1)

    x1 = _D1(acc1, cnt1r, hx, ntf1, rel_W1, root_W1, b1p)

    acc2a, acc2b, cnt2 = _L2(
        x1[:, :H], x1[:, H:],
        edge_index2[0], edge_index2[1], e_id2, edge_type,
        z64b, z8b, ones_in)

    acc2 = jnp.concatenate([acc2a[:ETY * S2].reshape(ETY, S2, H),
                            acc2b[:ETY * S2].reshape(ETY, S2, H)], axis=-1)
    cnt2r = cnt2[:ETY * S2, :1].reshape(ETY, S2, 1)
    ntf2 = node_type[:S2].astype(F32)[:, None]
    b2p = jnp.zeros((8, IN_C), F32).at[:NTY].set(root_b2)

    return _D2(acc2, cnt2r, x1[:S2], ntf2, rel_W2, root_W2, b2p)


# speedup vs baseline: 12.0910x; 1.2197x over previous
import functools

import jax
import jax.numpy as jnp
from jax import lax
from jax.experimental import pallas as pl
from jax.experimental.pallas import tpu as pltpu
from jax.experimental.pallas import tpu_sc as plsc

F32 = jnp.float32
I32 = jnp.int32

IN_C = 128
NTY = 2
ETY = 4
N0 = 10000
S1 = 5000
S2 = 2000
E1 = 320000
E2 = 160000
ETOT = 480000
NX0 = 6000

NC = 2
NS = 16
L = 16
K = 80
H = 64
HROWS = 5120
XROWS = 2048
AP1 = 20480
AP2 = 8192


def _sc_layer1(tbl_lo, tbl_hi, nt_h, li_h, src_h, dst_h, eid_h, et_h,
               z64, z8, ones_in,
               h_lo, h_hi, acc_a, acc_b, cnt_o,
               acc_s, cnt_s, h_s,
               srcv, dstv, eidv, keyv, etv, rowsv, onesv, ntv, liv, cidxv,
               sem):
    rpt = AP1 // NS
    ept = E1 // NS
    nch = ept // K
    hpt = HROWS // NS

    c = lax.axis_index("c")
    s = lax.axis_index("s")
    is0 = c == 0

    ab = s * rpt
    pltpu.sync_copy(z64, acc_s.at[pl.ds(ab, rpt)])

    @pl.when(is0)
    def _():
        pltpu.sync_copy(z8, cnt_s.at[pl.ds(ab, rpt)])

    pltpu.sync_copy(ones_in, onesv)

    for i in range(hpt // K):
        base = s * hpt + i * K
        pltpu.sync_copy(nt_h.at[pl.ds(base, K)], ntv)
        pltpu.sync_copy(li_h.at[pl.ds(base, K)], liv)
        for j in range(K // L):
            nt16 = ntv[pl.ds(j * L, L)]
            li16 = liv[pl.ds(j * L, L)]
            cidxv[pl.ds(j * L, L)] = jnp.where(nt16 == 0, li16, li16 + NX0)

        @pl.when(is0)
        def _():
            pltpu.async_copy(tbl_lo.at[cidxv], rowsv, sem).wait()
            pltpu.sync_copy(rowsv, h_lo.at[pl.ds(base, K)])

        @pl.when(~is0)
        def _():
            pltpu.async_copy(tbl_hi.at[cidxv], rowsv, sem).wait()
            pltpu.sync_copy(rowsv, h_hi.at[pl.ds(base, K)])

        pltpu.sync_copy(rowsv, h_s.at[pl.ds(base, K)])

    plsc.subcore_barrier()

    def edge_chunk(i, carry):
        base = s * ept + i * K
        pltpu.sync_copy(src_h.at[pl.ds(base, K)], srcv)
        pltpu.sync_copy(dst_h.at[pl.ds(base, K)], dstv)
        pltpu.sync_copy(eid_h.at[pl.ds(base, K)], eidv)
        pltpu.async_copy(et_h.at[eidv], etv, sem).wait()
        for j in range(K // L):
            et16 = etv[pl.ds(j * L, L)]
            d16 = dstv[pl.ds(j * L, L)]
            keyv[pl.ds(j * L, L)] = et16 * S1 + d16

        pltpu.async_copy(h_s.at[srcv], rowsv, sem).wait()
        pltpu.sync_copy(rowsv, acc_s.at[keyv], add=True)

        @pl.when(is0)
        def _():
            pltpu.sync_copy(onesv, cnt_s.at[keyv], add=True)

        return carry

    lax.fori_loop(0, nch, edge_chunk, 0)
    plsc.subcore_barrier()

    @pl.when(is0)
    def _():
        pltpu.sync_copy(acc_s.at[pl.ds(ab, rpt)], acc_a.at[pl.ds(ab, rpt)])
        pltpu.sync_copy(cnt_s.at[pl.ds(ab, rpt)], cnt_o.at[pl.ds(ab, rpt)])

    @pl.when(~is0)
    def _():
        pltpu.sync_copy(acc_s.at[pl.ds(ab, rpt)], acc_b.at[pl.ds(ab, rpt)])


def _sc_layer2(x_lo, x_hi, src_h, dst_h, eid_h, et_h, z64, z8, ones_in,
               acc_a, acc_b, cnt_o,
               acc_s, cnt_s, x_s, et_s,
               srcv, dstv, eidv, keyv, etv, rowsv, onesv,
               sem):
    rpt = AP2 // NS
    ept = E2 // NS
    nch = ept // K
    xpt = XROWS // NS
    etpt = ETOT // NS

    c = lax.axis_index("c")
    s = lax.axis_index("s")
    is0 = c == 0

    ab = s * rpt
    pltpu.sync_copy(z64, acc_s.at[pl.ds(ab, rpt)])

    @pl.when(is0)
    def _():
        pltpu.sync_copy(z8, cnt_s.at[pl.ds(ab, rpt)])

    @pl.when(is0)
    def _():
        pltpu.sync_copy(x_lo.at[pl.ds(s * xpt, xpt)], x_s.at[pl.ds(s * xpt, xpt)])

    @pl.when(~is0)
    def _():
        pltpu.sync_copy(x_hi.at[pl.ds(s * xpt, xpt)], x_s.at[pl.ds(s * xpt, xpt)])

    pltpu.sync_copy(et_h.at[pl.ds(s * etpt, etpt)], et_s.at[pl.ds(s * etpt, etpt)])
    pltpu.sync_copy(ones_in, onesv)
    plsc.subcore_barrier()

    def edge_chunk(i, carry):
        base = s * ept + i * K
        pltpu.sync_copy(src_h.at[pl.ds(base, K)], srcv)
        pltpu.sync_copy(dst_h.at[pl.ds(base, K)], dstv)
        pltpu.sync_copy(eid_h.at[pl.ds(base, K)], eidv)
        pltpu.async_copy(et_s.at[eidv], etv, sem).wait()
        for j in range(K // L):
            et16 = etv[pl.ds(j * L, L)]
            d16 = dstv[pl.ds(j * L, L)]
            keyv[pl.ds(j * L, L)] = et16 * S2 + d16

        pltpu.async_copy(x_s.at[srcv], rowsv, sem).wait()
        pltpu.sync_copy(rowsv, acc_s.at[keyv], add=True)

        @pl.when(is0)
        def _():
            pltpu.sync_copy(onesv, cnt_s.at[keyv], add=True)

        return carry

    lax.fori_loop(0, nch, edge_chunk, 0)
    plsc.subcore_barrier()

    @pl.when(is0)
    def _():
        pltpu.sync_copy(acc_s.at[pl.ds(ab, rpt)], acc_a.at[pl.ds(ab, rpt)])
        pltpu.sync_copy(cnt_s.at[pl.ds(ab, rpt)], cnt_o.at[pl.ds(ab, rpt)])

    @pl.when(~is0)
    def _():
        pltpu.sync_copy(acc_s.at[pl.ds(ab, rpt)], acc_b.at[pl.ds(ab, rpt)])


def _make_l1():
    A = AP1
    mesh = plsc.VectorSubcoreMesh(core_axis_name="c", subcore_axis_name="s")
    return pl.kernel(
        _sc_layer1,
        compiler_params=pltpu.CompilerParams(use_tc_tiling_on_sc=False),
        out_type=[
            jax.ShapeDtypeStruct((HROWS, H), F32),
            jax.ShapeDtypeStruct((HROWS, H), F32),
            jax.ShapeDtypeStruct((A, H), F32),
            jax.ShapeDtypeStruct((A, H), F32),
            jax.ShapeDtypeStruct((A, 8), F32),
        ],
        mesh=mesh,
        scratch_types=[
            pltpu.VMEM_SHARED((A, H), F32),
            pltpu.VMEM_SHARED((A, 8), F32),
            pltpu.VMEM_SHARED((HROWS, H), F32),
            pltpu.VMEM((K,), I32),
            pltpu.VMEM((K,), I32),
            pltpu.VMEM((K,), I32),
            pltpu.VMEM((K,), I32),
            pltpu.VMEM((K,), I32),
            pltpu.VMEM((K, H), F32),
            pltpu.VMEM((K, 8), F32),
            pltpu.VMEM((K,), I32),
            pltpu.VMEM((K,), I32),
            pltpu.VMEM((K,), I32),
            pltpu.SemaphoreType.DMA,
        ],
    )


def _make_l2():
    A = AP2
    mesh = plsc.VectorSubcoreMesh(core_axis_name="c", subcore_axis_name="s")
    return pl.kernel(
        _sc_layer2,
        compiler_params=pltpu.CompilerParams(use_tc_tiling_on_sc=False),
        out_type=[
            jax.ShapeDtypeStruct((A, H), F32),
            jax.ShapeDtypeStruct((A, H), F32),
            jax.ShapeDtypeStruct((A, 8), F32),
        ],
        mesh=mesh,
        scratch_types=[
            pltpu.VMEM_SHARED((A, H), F32),
            pltpu.VMEM_SHARED((A, 8), F32),
            pltpu.VMEM_SHARED((XROWS, H), F32),
            pltpu.VMEM_SHARED((ETOT,), I32),
            pltpu.VMEM((K,), I32),
            pltpu.VMEM((K,), I32),
            pltpu.VMEM((K,), I32),
            pltpu.VMEM((K,), I32),
            pltpu.VMEM((K,), I32),
            pltpu.VMEM((K, H), F32),
            pltpu.VMEM((K, 8), F32),
            pltpu.SemaphoreType.DMA,
        ],
    )


def _dense_body(acc_ref, cnt_ref, x_ref, ntf_ref, w_ref, r_ref, b_ref, o_ref,
                *, last_t, relu, logsm):
    t = pl.program_id(1)

    @pl.when(t == 0)
    def _():
        x = x_ref[...]
        m0 = ntf_ref[...] == 0.0
        r0 = jnp.dot(x, r_ref[0], preferred_element_type=F32) + b_ref[0, :]
        r1 = jnp.dot(x, r_ref[1], preferred_element_type=F32) + b_ref[1, :]
        o_ref[...] = jnp.where(m0, r0, r1)

    inv = 1.0 / jnp.maximum(cnt_ref[0], 1.0)
    a = acc_ref[0] * inv
    o_ref[...] += jnp.dot(a, w_ref[0], preferred_element_type=F32)

    @pl.when(t == last_t)
    def _():
        y = o_ref[...]
        if relu:
            o_ref[...] = jnp.maximum(y, 0.0)
        if logsm:
            m = jnp.max(y, axis=-1, keepdims=True)
            e = jnp.exp(y - m)
            o_ref[...] = y - m - jnp.log(jnp.sum(e, axis=-1, keepdims=True))


def _make_dense(S, blk, relu, logsm):
    rb = S // blk
    body = functools.partial(_dense_body, last_t=ETY - 1, relu=relu,
                             logsm=logsm)
    return pl.pallas_call(
        body,
        grid=(rb, ETY),
        in_specs=[
            pl.BlockSpec((1, blk, IN_C), lambda i, t: (t, i, 0)),
            pl.BlockSpec((1, blk, 1), lambda i, t: (t, i, 0)),
            pl.BlockSpec((blk, IN_C), lambda i, t: (i, 0)),
            pl.BlockSpec((blk, 1), lambda i, t: (i, 0)),
            pl.BlockSpec((1, IN_C, IN_C), lambda i, t: (t, 0, 0)),
            pl.BlockSpec((NTY, IN_C, IN_C), lambda i, t: (0, 0, 0)),
            pl.BlockSpec((8, IN_C), lambda i, t: (0, 0)),
        ],
        out_specs=pl.BlockSpec((blk, IN_C), lambda i, t: (i, 0)),
        out_shape=jax.ShapeDtypeStruct((S, IN_C), F32),
    )


_L1 = _make_l1()
_L2 = _make_l2()
_D1 = _make_dense(S1, 1000, relu=True, logsm=False)
_D2 = _make_dense(S2, 1000, relu=False, logsm=True)


def kernel(n_id, x0, emb1, edge_index1, e_id1, edge_index2, e_id2, edge_type,
           node_type, local_node_idx, rel_W1, root_W1, root_b1, rel_W2,
           root_W2, root_b2):
    tbl = jnp.concatenate([x0, emb1], axis=0)
    tbl_lo = tbl[:, :H]
    tbl_hi = tbl[:, H:]
    z64 = jnp.zeros((AP1 // NS, H), F32)
    z8 = jnp.zeros((AP1 // NS, 8), F32)
    z64b = jnp.zeros((AP2 // NS, H), F32)
    z8b = jnp.zeros((AP2 // NS, 8), F32)
    ones_in = jnp.ones((K, 8), F32)

    h_lo, h_hi, acc1a, acc1b, cnt1 = _L1(
        tbl_lo, tbl_hi, node_type, local_node_idx,
        edge_index1[0], edge_index1[1], e_id1, edge_type,
        z64, z8, ones_in)

    acc1 = jnp.concatenate([acc1a[:ETY * S1].reshape(ETY, S1, H),
                            acc1b[:ETY * S1].reshape(ETY, S1, H)], axis=-1)
    cnt1r = cnt1[:ETY * S1, :1].reshape(ETY, S1, 1)
    hx = jnp.concatenate([h_lo[:S1], h_hi[:S1]], axis=-1)
    ntf1 = node_type[:S1].astype(F32)[:, None]
    b1p = jnp.zeros((8, IN_C), F32).at[:NTY].set(root_b1)

    x1 = _D1(acc1, cnt1r, hx, ntf1, rel_W1, root_W1, b1p)

    acc2a, acc2b, cnt2 = _L2(
        x1[:, :H], x1[:, H:],
        edge_index2[0], edge_index2[1], e_id2, edge_type,
        z64b, z8b, ones_in)

    acc2 = jnp.concatenate([acc2a[:ETY * S2].reshape(ETY, S2, H),
                            acc2b[:ETY * S2].reshape(ETY, S2, H)], axis=-1)
    cnt2r = cnt2[:ETY * S2, :1].reshape(ETY, S2, 1)
    ntf2 = node_type[:S2].astype(F32)[:, None]
    b2p = jnp.zeros((8, IN_C), F32).at[:NTY].set(root_b2)

    return _D2(acc2, cnt2r, x1[:S2], ntf2, rel_W2, root_W2, b2p)


# speedup vs baseline: 19.9543x; 1.6503x over previous
import functools

import jax
import jax.numpy as jnp
from jax import lax
from jax.experimental import pallas as pl
from jax.experimental.pallas import tpu as pltpu
from jax.experimental.pallas import tpu_sc as plsc

F32 = jnp.float32
I32 = jnp.int32

IN_C = 128
NTY = 2
ETY = 4
N0 = 10000
S1 = 5000
S2 = 2000
E1 = 320000
E2 = 160000
ETOT = 480000
NX0 = 6000

NC = 2
NS = 16
L = 16
K = 80
H = 64
HROWS = 5120
XROWS = 2048
SCK = 4000
AP1 = 20480
AP2 = 8192


def _sc_layer1(tbl_lo, tbl_hi, nt_h, li_h, src_h, dst_h, eid_h, et_h,
               z64, z8, ones_in,
               h_lo, h_hi, acc_a, acc_b, cnt_o,
               acc_s, cnt_s, h_s,
               src_a, dst_a, eid_a, keyv, etv, rowsv, onesv, ntv, liv, cidxv,
               sem):
    rpt = AP1 // NS
    ept = E1 // NS
    nch = ept // K
    hpt = HROWS // NS

    c = lax.axis_index("c")
    s = lax.axis_index("s")
    is0 = c == 0

    ab = s * rpt
    pltpu.sync_copy(z64, acc_s.at[pl.ds(ab, rpt)])

    @pl.when(is0)
    def _():
        pltpu.sync_copy(z8, cnt_s.at[pl.ds(ab, rpt)])

    pltpu.sync_copy(ones_in, onesv)

    for i in range(hpt // K):
        base = s * hpt + i * K
        pltpu.sync_copy(nt_h.at[pl.ds(base, K)], ntv)
        pltpu.sync_copy(li_h.at[pl.ds(base, K)], liv)
        for j in range(K // L):
            nt16 = ntv[pl.ds(j * L, L)]
            li16 = liv[pl.ds(j * L, L)]
            cidxv[pl.ds(j * L, L)] = jnp.where(nt16 == 0, li16, li16 + NX0)

        @pl.when(is0)
        def _():
            pltpu.async_copy(tbl_lo.at[cidxv], rowsv, sem).wait()
            pltpu.sync_copy(rowsv, h_lo.at[pl.ds(base, K)])

        @pl.when(~is0)
        def _():
            pltpu.async_copy(tbl_hi.at[cidxv], rowsv, sem).wait()
            pltpu.sync_copy(rowsv, h_hi.at[pl.ds(base, K)])

        pltpu.sync_copy(rowsv, h_s.at[pl.ds(base, K)])

    plsc.subcore_barrier()

    def super_chunk(u, carry):
        ub = s * ept + u * SCK
        pltpu.sync_copy(src_h.at[pl.ds(ub, SCK)], src_a)
        pltpu.sync_copy(dst_h.at[pl.ds(ub, SCK)], dst_a)
        pltpu.sync_copy(eid_h.at[pl.ds(ub, SCK)], eid_a)

        def edge_chunk(i, carry2):
            base = i * K
            pltpu.async_copy(et_h.at[eid_a.at[pl.ds(base, K)]], etv,
                             sem).wait()
            for j in range(K // L):
                et16 = etv[pl.ds(j * L, L)]
                d16 = dst_a[pl.ds(base + j * L, L)]
                keyv[pl.ds(j * L, L)] = et16 * S1 + d16

            pltpu.async_copy(h_s.at[src_a.at[pl.ds(base, K)]], rowsv,
                             sem).wait()
            pltpu.sync_copy(rowsv, acc_s.at[keyv], add=True)

            @pl.when(is0)
            def _():
                pltpu.sync_copy(onesv, cnt_s.at[keyv], add=True)

            return carry2

        lax.fori_loop(0, SCK // K, edge_chunk, 0)
        return carry

    lax.fori_loop(0, ept // SCK, super_chunk, 0)
    plsc.subcore_barrier()

    @pl.when(is0)
    def _():
        pltpu.sync_copy(acc_s.at[pl.ds(ab, rpt)], acc_a.at[pl.ds(ab, rpt)])
        pltpu.sync_copy(cnt_s.at[pl.ds(ab, rpt)], cnt_o.at[pl.ds(ab, rpt)])

    @pl.when(~is0)
    def _():
        pltpu.sync_copy(acc_s.at[pl.ds(ab, rpt)], acc_b.at[pl.ds(ab, rpt)])


def _sc_layer2(x_lo, x_hi, src_h, dst_h, eid_h, et_h, z64, z8, ones_in,
               acc_a, acc_b, cnt_o,
               acc_s, cnt_s, x_s, et_s,
               src_a, dst_a, eid_a, keyv, etv, rowsv, onesv,
               sem):
    rpt = AP2 // NS
    ept = E2 // NS
    nch = ept // K
    xpt = XROWS // NS
    etpt = ETOT // NS

    c = lax.axis_index("c")
    s = lax.axis_index("s")
    is0 = c == 0

    ab = s * rpt
    pltpu.sync_copy(z64, acc_s.at[pl.ds(ab, rpt)])

    @pl.when(is0)
    def _():
        pltpu.sync_copy(z8, cnt_s.at[pl.ds(ab, rpt)])

    @pl.when(is0)
    def _():
        pltpu.sync_copy(x_lo.at[pl.ds(s * xpt, xpt)], x_s.at[pl.ds(s * xpt, xpt)])

    @pl.when(~is0)
    def _():
        pltpu.sync_copy(x_hi.at[pl.ds(s * xpt, xpt)], x_s.at[pl.ds(s * xpt, xpt)])

    pltpu.sync_copy(et_h.at[pl.ds(s * etpt, etpt)], et_s.at[pl.ds(s * etpt, etpt)])
    pltpu.sync_copy(ones_in, onesv)

    eb = s * ept
    pltpu.sync_copy(src_h.at[pl.ds(eb, ept)], src_a)
    pltpu.sync_copy(dst_h.at[pl.ds(eb, ept)], dst_a)
    pltpu.sync_copy(eid_h.at[pl.ds(eb, ept)], eid_a)
    plsc.subcore_barrier()

    def edge_chunk(i, carry):
        base = i * K
        pltpu.async_copy(et_s.at[eid_a.at[pl.ds(base, K)]], etv, sem).wait()
        for j in range(K // L):
            et16 = etv[pl.ds(j * L, L)]
            d16 = dst_a[pl.ds(base + j * L, L)]
            keyv[pl.ds(j * L, L)] = et16 * S2 + d16

        pltpu.async_copy(x_s.at[src_a.at[pl.ds(base, K)]], rowsv, sem).wait()
        pltpu.sync_copy(rowsv, acc_s.at[keyv], add=True)

        @pl.when(is0)
        def _():
            pltpu.sync_copy(onesv, cnt_s.at[keyv], add=True)

        return carry

    lax.fori_loop(0, nch, edge_chunk, 0)
    plsc.subcore_barrier()

    @pl.when(is0)
    def _():
        pltpu.sync_copy(acc_s.at[pl.ds(ab, rpt)], acc_a.at[pl.ds(ab, rpt)])
        pltpu.sync_copy(cnt_s.at[pl.ds(ab, rpt)], cnt_o.at[pl.ds(ab, rpt)])

    @pl.when(~is0)
    def _():
        pltpu.sync_copy(acc_s.at[pl.ds(ab, rpt)], acc_b.at[pl.ds(ab, rpt)])


def _make_l1():
    A = AP1
    mesh = plsc.VectorSubcoreMesh(core_axis_name="c", subcore_axis_name="s")
    return pl.kernel(
        _sc_layer1,
        compiler_params=pltpu.CompilerParams(use_tc_tiling_on_sc=False),
        out_type=[
            jax.ShapeDtypeStruct((HROWS, H), F32),
            jax.ShapeDtypeStruct((HROWS, H), F32),
            jax.ShapeDtypeStruct((A, H), F32),
            jax.ShapeDtypeStruct((A, H), F32),
            jax.ShapeDtypeStruct((A, 8), F32),
        ],
        mesh=mesh,
        scratch_types=[
            pltpu.VMEM_SHARED((A, H), F32),
            pltpu.VMEM_SHARED((A, 8), F32),
            pltpu.VMEM_SHARED((HROWS, H), F32),
            pltpu.VMEM((SCK,), I32),
            pltpu.VMEM((SCK,), I32),
            pltpu.VMEM((SCK,), I32),
            pltpu.VMEM((K,), I32),
            pltpu.VMEM((K,), I32),
            pltpu.VMEM((K, H), F32),
            pltpu.VMEM((K, 8), F32),
            pltpu.VMEM((K,), I32),
            pltpu.VMEM((K,), I32),
            pltpu.VMEM((K,), I32),
            pltpu.SemaphoreType.DMA,
        ],
    )


def _make_l2():
    A = AP2
    mesh = plsc.VectorSubcoreMesh(core_axis_name="c", subcore_axis_name="s")
    return pl.kernel(
        _sc_layer2,
        compiler_params=pltpu.CompilerParams(use_tc_tiling_on_sc=False),
        out_type=[
            jax.ShapeDtypeStruct((A, H), F32),
            jax.ShapeDtypeStruct((A, H), F32),
            jax.ShapeDtypeStruct((A, 8), F32),
        ],
        mesh=mesh,
        scratch_types=[
            pltpu.VMEM_SHARED((A, H), F32),
            pltpu.VMEM_SHARED((A, 8), F32),
            pltpu.VMEM_SHARED((XROWS, H), F32),
            pltpu.VMEM_SHARED((ETOT,), I32),
            pltpu.VMEM((E2 // NS,), I32),
            pltpu.VMEM((E2 // NS,), I32),
            pltpu.VMEM((E2 // NS,), I32),
            pltpu.VMEM((K,), I32),
            pltpu.VMEM((K,), I32),
            pltpu.VMEM((K, H), F32),
            pltpu.VMEM((K, 8), F32),
            pltpu.SemaphoreType.DMA,
        ],
    )


def _dense_body(acc_ref, cnt_ref, x_ref, ntf_ref, w_ref, r_ref, b_ref, o_ref,
                *, last_t, relu, logsm):
    t = pl.program_id(1)

    @pl.when(t == 0)
    def _():
        x = x_ref[...]
        m0 = ntf_ref[...] == 0.0
        r0 = jnp.dot(x, r_ref[0], preferred_element_type=F32) + b_ref[0, :]
        r1 = jnp.dot(x, r_ref[1], preferred_element_type=F32) + b_ref[1, :]
        o_ref[...] = jnp.where(m0, r0, r1)

    inv = 1.0 / jnp.maximum(cnt_ref[0], 1.0)
    a = acc_ref[0] * inv
    o_ref[...] += jnp.dot(a, w_ref[0], preferred_element_type=F32)

    @pl.when(t == last_t)
    def _():
        y = o_ref[...]
        if relu:
            o_ref[...] = jnp.maximum(y, 0.0)
        if logsm:
            m = jnp.max(y, axis=-1, keepdims=True)
            e = jnp.exp(y - m)
            o_ref[...] = y - m - jnp.log(jnp.sum(e, axis=-1, keepdims=True))


def _make_dense(S, blk, relu, logsm):
    rb = S // blk
    body = functools.partial(_dense_body, last_t=ETY - 1, relu=relu,
                             logsm=logsm)
    return pl.pallas_call(
        body,
        grid=(rb, ETY),
        in_specs=[
            pl.BlockSpec((1, blk, IN_C), lambda i, t: (t, i, 0)),
            pl.BlockSpec((1, blk, 1), lambda i, t: (t, i, 0)),
            pl.BlockSpec((blk, IN_C), lambda i, t: (i, 0)),
            pl.BlockSpec((blk, 1), lambda i, t: (i, 0)),
            pl.BlockSpec((1, IN_C, IN_C), lambda i, t: (t, 0, 0)),
            pl.BlockSpec((NTY, IN_C, IN_C), lambda i, t: (0, 0, 0)),
            pl.BlockSpec((8, IN_C), lambda i, t: (0, 0)),
        ],
        out_specs=pl.BlockSpec((blk, IN_C), lambda i, t: (i, 0)),
        out_shape=jax.ShapeDtypeStruct((S, IN_C), F32),
    )


_L1 = _make_l1()
_L2 = _make_l2()
_D1 = _make_dense(S1, 1000, relu=True, logsm=False)
_D2 = _make_dense(S2, 1000, relu=False, logsm=True)


def kernel(n_id, x0, emb1, edge_index1, e_id1, edge_index2, e_id2, edge_type,
           node_type, local_node_idx, rel_W1, root_W1, root_b1, rel_W2,
           root_W2, root_b2):
    tbl = jnp.concatenate([x0, emb1], axis=0)
    tbl_lo = tbl[:, :H]
    tbl_hi = tbl[:, H:]
    z64 = jnp.zeros((AP1 // NS, H), F32)
    z8 = jnp.zeros((AP1 // NS, 8), F32)
    z64b = jnp.zeros((AP2 // NS, H), F32)
    z8b = jnp.zeros((AP2 // NS, 8), F32)
    ones_in = jnp.ones((K, 8), F32)

    h_lo, h_hi, acc1a, acc1b, cnt1 = _L1(
        tbl_lo, tbl_hi, node_type, local_node_idx,
        edge_index1[0], edge_index1[1], e_id1, edge_type,
        z64, z8, ones_in)

    acc1 = jnp.concatenate([acc1a[:ETY * S1].reshape(ETY, S1, H),
                            acc1b[:ETY * S1].reshape(ETY, S1, H)], axis=-1)
    cnt1r = cnt1[:ETY * S1, :1].reshape(ETY, S1, 1)
    hx = jnp.concatenate([h_lo[:S1], h_hi[:S1]], axis=-1)
    ntf1 = node_type[:S1].astype(F32)[:, None]
    b1p = jnp.zeros((8, IN_C), F32).at[:NTY].set(root_b1)

    x1 = _D1(acc1, cnt1r, hx, ntf1, rel_W1, root_W1, b1p)

    acc2a, acc2b, cnt2 = _L2(
        x1[:, :H], x1[:, H:],
        edge_index2[0], edge_index2[1], e_id2, edge_type,
        z64b, z8b, ones_in)

    acc2 = jnp.concatenate([acc2a[:ETY * S2].reshape(ETY, S2, H),
                            acc2b[:ETY * S2].reshape(ETY, S2, H)], axis=-1)
    cnt2r = cnt2[:ETY * S2, :1].reshape(ETY, S2, 1)
    ntf2 = node_type[:S2].astype(F32)[:, None]
    b2p = jnp.zeros((8, IN_C), F32).at[:NTY].set(root_b2)

    return _D2(acc2, cnt2r, x1[:S2], ntf2, rel_W2, root_W2, b2p)


# speedup vs baseline: 20.6664x; 1.0357x over previous
import functools

import jax
import jax.numpy as jnp
from jax import lax
from jax.experimental import pallas as pl
from jax.experimental.pallas import tpu as pltpu
from jax.experimental.pallas import tpu_sc as plsc

F32 = jnp.float32
I32 = jnp.int32

IN_C = 128
NTY = 2
ETY = 4
N0 = 10000
S1 = 5000
S2 = 2000
E1 = 320000
E2 = 160000
ETOT = 480000
NX0 = 6000

NC = 2
NS = 16
L = 16
K = 80
H = 64
HROWS = 5120
XROWS = 2048
SCK = 4000
AP1 = 20480
AP2 = 8192


def _sc_layer1(tbl_lo, tbl_hi, nt_h, li_h, src_h, dst_h, eid_h, et_h,
               z64, z8, ones_in,
               h_lo, h_hi, acc_a, acc_b, cnt_o,
               acc_s, cnt_s, h_s,
               src_a, dst_a, eid_a, keyv, etv, rowsv, onesv, ntv, liv, cidxv,
               sem):
    rpt = AP1 // NS
    ept = E1 // NS
    nch = ept // K
    hpt = HROWS // NS

    c = lax.axis_index("c")
    s = lax.axis_index("s")
    is0 = c == 0

    ab = s * rpt
    pltpu.sync_copy(z64, acc_s.at[pl.ds(ab, rpt)])

    @pl.when(is0)
    def _():
        pltpu.sync_copy(z8, cnt_s.at[pl.ds(ab, rpt)])

    pltpu.sync_copy(ones_in, onesv)

    for i in range(hpt // K):
        base = s * hpt + i * K
        pltpu.sync_copy(nt_h.at[pl.ds(base, K)], ntv)
        pltpu.sync_copy(li_h.at[pl.ds(base, K)], liv)
        for j in range(K // L):
            nt16 = ntv[pl.ds(j * L, L)]
            li16 = liv[pl.ds(j * L, L)]
            cidxv[pl.ds(j * L, L)] = jnp.where(nt16 == 0, li16, li16 + NX0)

        @pl.when(is0)
        def _():
            pltpu.async_copy(tbl_lo.at[cidxv], rowsv, sem).wait()
            pltpu.sync_copy(rowsv, h_lo.at[pl.ds(base, K)])

        @pl.when(~is0)
        def _():
            pltpu.async_copy(tbl_hi.at[cidxv], rowsv, sem).wait()
            pltpu.sync_copy(rowsv, h_hi.at[pl.ds(base, K)])

        pltpu.sync_copy(rowsv, h_s.at[pl.ds(base, K)])

    plsc.subcore_barrier()

    def super_chunk(u, carry):
        ub = s * ept + u * SCK
        pltpu.sync_copy(src_h.at[pl.ds(ub, SCK)], src_a)
        pltpu.sync_copy(dst_h.at[pl.ds(ub, SCK)], dst_a)
        pltpu.sync_copy(eid_h.at[pl.ds(ub, SCK)], eid_a)

        def edge_chunk(i, carry2):
            base = i * K
            pltpu.async_copy(et_h.at[eid_a.at[pl.ds(base, K)]], etv,
                             sem).wait()
            for j in range(K // L):
                et16 = etv[pl.ds(j * L, L)]
                d16 = dst_a[pl.ds(base + j * L, L)]
                keyv[pl.ds(j * L, L)] = et16 * S1 + d16

            pltpu.async_copy(h_s.at[src_a.at[pl.ds(base, K)]], rowsv,
                             sem).wait()
            pltpu.sync_copy(rowsv, acc_s.at[keyv], add=True)

            @pl.when(is0)
            def _():
                pltpu.sync_copy(onesv, cnt_s.at[keyv], add=True)

            return carry2

        lax.fori_loop(0, SCK // K, edge_chunk, 0)
        return carry

    lax.fori_loop(0, ept // SCK, super_chunk, 0)
    plsc.subcore_barrier()

    @pl.when(is0)
    def _():
        pltpu.sync_copy(acc_s.at[pl.ds(ab, rpt)], acc_a.at[pl.ds(ab, rpt)])
        pltpu.sync_copy(cnt_s.at[pl.ds(ab, rpt)], cnt_o.at[pl.ds(ab, rpt)])

    @pl.when(~is0)
    def _():
        pltpu.sync_copy(acc_s.at[pl.ds(ab, rpt)], acc_b.at[pl.ds(ab, rpt)])


def _sc_layer2(x_lo, x_hi, src_h, dst_h, eid_h, et_h, z64, z8, ones_in,
               acc_a, acc_b, cnt_o,
               acc_s, cnt_s, x_s, et_s,
               src_a, dst_a, eid_a, keyv, etv, rowsv, onesv,
               sem):
    rpt = AP2 // NS
    ept = E2 // NS
    nch = ept // K
    xpt = XROWS // NS
    etpt = ETOT // NS

    c = lax.axis_index("c")
    s = lax.axis_index("s")
    is0 = c == 0

    ab = s * rpt
    pltpu.sync_copy(z64, acc_s.at[pl.ds(ab, rpt)])

    @pl.when(is0)
    def _():
        pltpu.sync_copy(z8, cnt_s.at[pl.ds(ab, rpt)])

    @pl.when(is0)
    def _():
        pltpu.sync_copy(x_lo.at[pl.ds(s * xpt, xpt)], x_s.at[pl.ds(s * xpt, xpt)])

    @pl.when(~is0)
    def _():
        pltpu.sync_copy(x_hi.at[pl.ds(s * xpt, xpt)], x_s.at[pl.ds(s * xpt, xpt)])

    pltpu.sync_copy(et_h.at[pl.ds(s * etpt, etpt)], et_s.at[pl.ds(s * etpt, etpt)])
    pltpu.sync_copy(ones_in, onesv)

    eb = s * ept
    pltpu.sync_copy(src_h.at[pl.ds(eb, ept)], src_a)
    pltpu.sync_copy(dst_h.at[pl.ds(eb, ept)], dst_a)
    pltpu.sync_copy(eid_h.at[pl.ds(eb, ept)], eid_a)
    plsc.subcore_barrier()

    def edge_chunk(i, carry):
        base = i * K
        pltpu.async_copy(et_s.at[eid_a.at[pl.ds(base, K)]], etv, sem).wait()
        for j in range(K // L):
            et16 = etv[pl.ds(j * L, L)]
            d16 = dst_a[pl.ds(base + j * L, L)]
            keyv[pl.ds(j * L, L)] = et16 * S2 + d16

        pltpu.async_copy(x_s.at[src_a.at[pl.ds(base, K)]], rowsv, sem).wait()
        pltpu.sync_copy(rowsv, acc_s.at[keyv], add=True)

        @pl.when(is0)
        def _():
            pltpu.sync_copy(onesv, cnt_s.at[keyv], add=True)

        return carry

    lax.fori_loop(0, nch, edge_chunk, 0)
    plsc.subcore_barrier()

    @pl.when(is0)
    def _():
        pltpu.sync_copy(acc_s.at[pl.ds(ab, rpt)], acc_a.at[pl.ds(ab, rpt)])
        pltpu.sync_copy(cnt_s.at[pl.ds(ab, rpt)], cnt_o.at[pl.ds(ab, rpt)])

    @pl.when(~is0)
    def _():
        pltpu.sync_copy(acc_s.at[pl.ds(ab, rpt)], acc_b.at[pl.ds(ab, rpt)])


def _make_l1():
    A = AP1
    mesh = plsc.VectorSubcoreMesh(core_axis_name="c", subcore_axis_name="s")
    return pl.kernel(
        _sc_layer1,
        compiler_params=pltpu.CompilerParams(use_tc_tiling_on_sc=False),
        out_type=[
            jax.ShapeDtypeStruct((HROWS, H), F32),
            jax.ShapeDtypeStruct((HROWS, H), F32),
            jax.ShapeDtypeStruct((A, H), F32),
            jax.ShapeDtypeStruct((A, H), F32),
            jax.ShapeDtypeStruct((A, 8), F32),
        ],
        mesh=mesh,
        scratch_types=[
            pltpu.VMEM_SHARED((A, H), F32),
            pltpu.VMEM_SHARED((A, 8), F32),
            pltpu.VMEM_SHARED((HROWS, H), F32),
            pltpu.VMEM((SCK,), I32),
            pltpu.VMEM((SCK,), I32),
            pltpu.VMEM((SCK,), I32),
            pltpu.VMEM((K,), I32),
            pltpu.VMEM((K,), I32),
            pltpu.VMEM((K, H), F32),
            pltpu.VMEM((K, 8), F32),
            pltpu.VMEM((K,), I32),
            pltpu.VMEM((K,), I32),
            pltpu.VMEM((K,), I32),
            pltpu.SemaphoreType.DMA,
        ],
    )


def _make_l2():
    A = AP2
    mesh = plsc.VectorSubcoreMesh(core_axis_name="c", subcore_axis_name="s")
    return pl.kernel(
        _sc_layer2,
        compiler_params=pltpu.CompilerParams(use_tc_tiling_on_sc=False),
        out_type=[
            jax.ShapeDtypeStruct((A, H), F32),
            jax.ShapeDtypeStruct((A, H), F32),
            jax.ShapeDtypeStruct((A, 8), F32),
        ],
        mesh=mesh,
        scratch_types=[
            pltpu.VMEM_SHARED((A, H), F32),
            pltpu.VMEM_SHARED((A, 8), F32),
            pltpu.VMEM_SHARED((XROWS, H), F32),
            pltpu.VMEM_SHARED((ETOT,), I32),
            pltpu.VMEM((E2 // NS,), I32),
            pltpu.VMEM((E2 // NS,), I32),
            pltpu.VMEM((E2 // NS,), I32),
            pltpu.VMEM((K,), I32),
            pltpu.VMEM((K,), I32),
            pltpu.VMEM((K, H), F32),
            pltpu.VMEM((K, 8), F32),
            pltpu.SemaphoreType.DMA,
        ],
    )


def _dense_body(acc_lo, acc_hi, cnt_ref, xl_ref, xh_ref, ntf_ref, w_ref,
                r_ref, b_ref, *outs, last_t, relu, logsm, split_out):
    t = pl.program_id(1)

    @pl.when(t == 0)
    def _():
        xl = xl_ref[...]
        xh = xh_ref[...]
        m0 = ntf_ref[...] == 0.0
        r0 = (jnp.dot(xl, r_ref[0, :H], preferred_element_type=F32) +
              jnp.dot(xh, r_ref[0, H:], preferred_element_type=F32) +
              b_ref[0, :])
        r1 = (jnp.dot(xl, r_ref[1, :H], preferred_element_type=F32) +
              jnp.dot(xh, r_ref[1, H:], preferred_element_type=F32) +
              b_ref[1, :])
        y = jnp.where(m0, r0, r1)
        if split_out:
            outs[0][...] = y[:, :H]
            outs[1][...] = y[:, H:]
        else:
            outs[0][...] = y

    inv = 1.0 / jnp.maximum(cnt_ref[:, :1], 1.0)
    al = acc_lo[...] * inv
    ah = acc_hi[...] * inv
    y = (jnp.dot(al, w_ref[0, :H], preferred_element_type=F32) +
         jnp.dot(ah, w_ref[0, H:], preferred_element_type=F32))
    if split_out:
        outs[0][...] += y[:, :H]
        outs[1][...] += y[:, H:]
    else:
        outs[0][...] += y

    @pl.when(t == last_t)
    def _():
        if relu:
            for o in outs:
                o[...] = jnp.maximum(o[...], 0.0)
        if logsm:
            y2 = outs[0][...]
            m = jnp.max(y2, axis=-1, keepdims=True)
            e = jnp.exp(y2 - m)
            outs[0][...] = y2 - m - jnp.log(
                jnp.sum(e, axis=-1, keepdims=True))


def _make_dense(S, blk, relu, logsm, split_out):
    rb = S // blk
    body = functools.partial(_dense_body, last_t=ETY - 1, relu=relu,
                             logsm=logsm, split_out=split_out)
    amap = lambda i, t: (t * rb + i, 0)
    if split_out:
        out_specs = [pl.BlockSpec((blk, H), lambda i, t: (i, 0)),
                     pl.BlockSpec((blk, H), lambda i, t: (i, 0))]
        out_shape = [jax.ShapeDtypeStruct((S, H), F32),
                     jax.ShapeDtypeStruct((S, H), F32)]
    else:
        out_specs = pl.BlockSpec((blk, IN_C), lambda i, t: (i, 0))
        out_shape = jax.ShapeDtypeStruct((S, IN_C), F32)
    return pl.pallas_call(
        body,
        grid=(rb, ETY),
        in_specs=[
            pl.BlockSpec((blk, H), amap),
            pl.BlockSpec((blk, H), amap),
            pl.BlockSpec((blk, 8), amap),
            pl.BlockSpec((blk, H), lambda i, t: (i, 0)),
            pl.BlockSpec((blk, H), lambda i, t: (i, 0)),
            pl.BlockSpec((blk, 1), lambda i, t: (i, 0)),
            pl.BlockSpec((1, IN_C, IN_C), lambda i, t: (t, 0, 0)),
            pl.BlockSpec((NTY, IN_C, IN_C), lambda i, t: (0, 0, 0)),
            pl.BlockSpec((8, IN_C), lambda i, t: (0, 0)),
        ],
        out_specs=out_specs,
        out_shape=out_shape,
    )


_L1 = _make_l1()
_L2 = _make_l2()
_D1 = _make_dense(S1, 1000, relu=True, logsm=False, split_out=True)
_D2 = _make_dense(S2, 1000, relu=False, logsm=True, split_out=False)


def kernel(n_id, x0, emb1, edge_index1, e_id1, edge_index2, e_id2, edge_type,
           node_type, local_node_idx, rel_W1, root_W1, root_b1, rel_W2,
           root_W2, root_b2):
    tbl = jnp.concatenate([x0, emb1], axis=0)
    tbl_lo = tbl[:, :H]
    tbl_hi = tbl[:, H:]
    z64 = jnp.zeros((AP1 // NS, H), F32)
    z8 = jnp.zeros((AP1 // NS, 8), F32)
    z64b = jnp.zeros((AP2 // NS, H), F32)
    z8b = jnp.zeros((AP2 // NS, 8), F32)
    ones_in = jnp.ones((K, 8), F32)

    h_lo, h_hi, acc1a, acc1b, cnt1 = _L1(
        tbl_lo, tbl_hi, node_type, local_node_idx,
        edge_index1[0], edge_index1[1], e_id1, edge_type,
        z64, z8, ones_in)

    ntf1 = node_type[:S1].astype(F32)[:, None]
    b1p = jnp.zeros((8, IN_C), F32).at[:NTY].set(root_b1)

    x1_lo, x1_hi = _D1(acc1a, acc1b, cnt1, h_lo, h_hi, ntf1,
                       rel_W1, root_W1, b1p)

    acc2a, acc2b, cnt2 = _L2(
        x1_lo, x1_hi,
        edge_index2[0], edge_index2[1], e_id2, edge_type,
        z64b, z8b, ones_in)

    ntf2 = node_type[:S2].astype(F32)[:, None]
    b2p = jnp.zeros((8, IN_C), F32).at[:NTY].set(root_b2)

    return _D2(acc2a, acc2b, cnt2, x1_lo, x1_hi, ntf2, rel_W2, root_W2, b2p)


# speedup vs baseline: 26.2886x; 1.2720x over previous
import functools

import jax
import jax.numpy as jnp
from jax import lax
from jax.experimental import pallas as pl
from jax.experimental.pallas import tpu as pltpu
from jax.experimental.pallas import tpu_sc as plsc

F32 = jnp.float32
I32 = jnp.int32

IN_C = 128
NTY = 2
ETY = 4
N0 = 10000
S1 = 5000
S2 = 2000
E1 = 320000
E2 = 160000
ETOT = 480000
NX0 = 6000

NC = 2
NS = 16
L = 16
K = 80
H = 64
HROWS = 5120
XROWS = 2048
SCK = 4000
AP1 = 20480
AP2 = 8192


def _sc_layer1(tbl_lo, tbl_hi, nt_h, li_h, src_h, dst_h, eid_h, et_h,
               z64, z8, ones_in,
               h_lo, h_hi, acc_a, acc_b, cnt_o,
               acc_s, cnt_s, h_s,
               src_a, dst_a, eid_a, keyv, etv0, etv1, rowsv, onesv,
               ntv, liv, cidxv,
               sem, semE0, semE1):
    rpt = AP1 // NS
    ept = E1 // NS
    nch = ept // K
    hpt = HROWS // NS

    c = lax.axis_index("c")
    s = lax.axis_index("s")
    is0 = c == 0

    ab = s * rpt
    pltpu.sync_copy(z64, acc_s.at[pl.ds(ab, rpt)])

    @pl.when(is0)
    def _():
        pltpu.sync_copy(z8, cnt_s.at[pl.ds(ab, rpt)])

    pltpu.sync_copy(ones_in, onesv)

    for i in range(hpt // K):
        base = s * hpt + i * K
        pltpu.sync_copy(nt_h.at[pl.ds(base, K)], ntv)
        pltpu.sync_copy(li_h.at[pl.ds(base, K)], liv)
        for j in range(K // L):
            nt16 = ntv[pl.ds(j * L, L)]
            li16 = liv[pl.ds(j * L, L)]
            cidxv[pl.ds(j * L, L)] = jnp.where(nt16 == 0, li16, li16 + NX0)

        @pl.when(is0)
        def _():
            pltpu.async_copy(tbl_lo.at[cidxv], rowsv, sem).wait()
            pltpu.sync_copy(rowsv, h_lo.at[pl.ds(base, K)])

        @pl.when(~is0)
        def _():
            pltpu.async_copy(tbl_hi.at[cidxv], rowsv, sem).wait()
            pltpu.sync_copy(rowsv, h_hi.at[pl.ds(base, K)])

        pltpu.sync_copy(rowsv, h_s.at[pl.ds(base, K)])

    plsc.subcore_barrier()

    nchs = SCK // K

    def super_chunk(u, carry):
        ub = s * ept + u * SCK
        pltpu.sync_copy(src_h.at[pl.ds(ub, SCK)], src_a)
        pltpu.sync_copy(dst_h.at[pl.ds(ub, SCK)], dst_a)
        pltpu.sync_copy(eid_h.at[pl.ds(ub, SCK)], eid_a)
        pltpu.async_copy(et_h.at[eid_a.at[pl.ds(0, K)]], etv0, semE0)

        def edge_chunk(i, carry2):
            base = i * K

            def do(cur, cursem, nxt, nxtsem):
                @pl.when(i + 1 < nchs)
                def _():
                    pltpu.async_copy(
                        et_h.at[eid_a.at[pl.ds(base + K, K)]], nxt, nxtsem)

                pltpu.make_async_copy(et_h.at[pl.ds(0, K)], cur,
                                      cursem).wait()
                for j in range(K // L):
                    et16 = cur[pl.ds(j * L, L)]
                    d16 = dst_a[pl.ds(base + j * L, L)]
                    keyv[pl.ds(j * L, L)] = et16 * S1 + d16

                pltpu.async_copy(h_s.at[src_a.at[pl.ds(base, K)]], rowsv,
                                 sem).wait()
                pltpu.sync_copy(rowsv, acc_s.at[keyv], add=True)

                @pl.when(is0)
                def _():
                    pltpu.sync_copy(onesv, cnt_s.at[keyv], add=True)

            @pl.when(i % 2 == 0)
            def _():
                do(etv0, semE0, etv1, semE1)

            @pl.when(i % 2 == 1)
            def _():
                do(etv1, semE1, etv0, semE0)

            return carry2

        lax.fori_loop(0, nchs, edge_chunk, 0)
        return carry

    lax.fori_loop(0, ept // SCK, super_chunk, 0)
    plsc.subcore_barrier()

    @pl.when(is0)
    def _():
        pltpu.sync_copy(acc_s.at[pl.ds(ab, rpt)], acc_a.at[pl.ds(ab, rpt)])
        pltpu.sync_copy(cnt_s.at[pl.ds(ab, rpt)], cnt_o.at[pl.ds(ab, rpt)])

    @pl.when(~is0)
    def _():
        pltpu.sync_copy(acc_s.at[pl.ds(ab, rpt)], acc_b.at[pl.ds(ab, rpt)])


def _sc_layer2(x_lo, x_hi, src_h, dst_h, eid_h, et_h, z64, z8, ones_in,
               acc_a, acc_b, cnt_o,
               acc_s, cnt_s, x_s, et_s,
               src_a, dst_a, eid_a, keyv, etv, rowsv, onesv,
               sem):
    rpt = AP2 // NS
    ept = E2 // NS
    nch = ept // K
    xpt = XROWS // NS
    etpt = ETOT // NS

    c = lax.axis_index("c")
    s = lax.axis_index("s")
    is0 = c == 0

    ab = s * rpt
    pltpu.sync_copy(z64, acc_s.at[pl.ds(ab, rpt)])

    @pl.when(is0)
    def _():
        pltpu.sync_copy(z8, cnt_s.at[pl.ds(ab, rpt)])

    @pl.when(is0)
    def _():
        pltpu.sync_copy(x_lo.at[pl.ds(s * xpt, xpt)], x_s.at[pl.ds(s * xpt, xpt)])

    @pl.when(~is0)
    def _():
        pltpu.sync_copy(x_hi.at[pl.ds(s * xpt, xpt)], x_s.at[pl.ds(s * xpt, xpt)])

    pltpu.sync_copy(et_h.at[pl.ds(s * etpt, etpt)], et_s.at[pl.ds(s * etpt, etpt)])
    pltpu.sync_copy(ones_in, onesv)

    eb = s * ept
    pltpu.sync_copy(src_h.at[pl.ds(eb, ept)], src_a)
    pltpu.sync_copy(dst_h.at[pl.ds(eb, ept)], dst_a)
    pltpu.sync_copy(eid_h.at[pl.ds(eb, ept)], eid_a)
    plsc.subcore_barrier()

    def edge_chunk(i, carry):
        base = i * K
        pltpu.async_copy(et_s.at[eid_a.at[pl.ds(base, K)]], etv, sem).wait()
        for j in range(K // L):
            et16 = etv[pl.ds(j * L, L)]
            d16 = dst_a[pl.ds(base + j * L, L)]
            keyv[pl.ds(j * L, L)] = et16 * S2 + d16

        pltpu.async_copy(x_s.at[src_a.at[pl.ds(base, K)]], rowsv, sem).wait()
        pltpu.sync_copy(rowsv, acc_s.at[keyv], add=True)

        @pl.when(is0)
        def _():
            pltpu.sync_copy(onesv, cnt_s.at[keyv], add=True)

        return carry

    lax.fori_loop(0, nch, edge_chunk, 0)
    plsc.subcore_barrier()

    @pl.when(is0)
    def _():
        pltpu.sync_copy(acc_s.at[pl.ds(ab, rpt)], acc_a.at[pl.ds(ab, rpt)])
        pltpu.sync_copy(cnt_s.at[pl.ds(ab, rpt)], cnt_o.at[pl.ds(ab, rpt)])

    @pl.when(~is0)
    def _():
        pltpu.sync_copy(acc_s.at[pl.ds(ab, rpt)], acc_b.at[pl.ds(ab, rpt)])


def _make_l1():
    A = AP1
    mesh = plsc.VectorSubcoreMesh(core_axis_name="c", subcore_axis_name="s")
    return pl.kernel(
        _sc_layer1,
        compiler_params=pltpu.CompilerParams(use_tc_tiling_on_sc=False),
        out_type=[
            jax.ShapeDtypeStruct((HROWS, H), F32),
            jax.ShapeDtypeStruct((HROWS, H), F32),
            jax.ShapeDtypeStruct((A, H), F32),
            jax.ShapeDtypeStruct((A, H), F32),
            jax.ShapeDtypeStruct((A, 8), F32),
        ],
        mesh=mesh,
        scratch_types=[
            pltpu.VMEM_SHARED((A, H), F32),
            pltpu.VMEM_SHARED((A, 8), F32),
            pltpu.VMEM_SHARED((HROWS, H), F32),
            pltpu.VMEM((SCK,), I32),
            pltpu.VMEM((SCK,), I32),
            pltpu.VMEM((SCK,), I32),
            pltpu.VMEM((K,), I32),
            pltpu.VMEM((K,), I32),
            pltpu.VMEM((K,), I32),
            pltpu.VMEM((K, H), F32),
            pltpu.VMEM((K, 8), F32),
            pltpu.VMEM((K,), I32),
            pltpu.VMEM((K,), I32),
            pltpu.VMEM((K,), I32),
            pltpu.SemaphoreType.DMA,
            pltpu.SemaphoreType.DMA,
            pltpu.SemaphoreType.DMA,
        ],
    )


def _make_l2():
    A = AP2
    mesh = plsc.VectorSubcoreMesh(core_axis_name="c", subcore_axis_name="s")
    return pl.kernel(
        _sc_layer2,
        compiler_params=pltpu.CompilerParams(use_tc_tiling_on_sc=False),
        out_type=[
            jax.ShapeDtypeStruct((A, H), F32),
            jax.ShapeDtypeStruct((A, H), F32),
            jax.ShapeDtypeStruct((A, 8), F32),
        ],
        mesh=mesh,
        scratch_types=[
            pltpu.VMEM_SHARED((A, H), F32),
            pltpu.VMEM_SHARED((A, 8), F32),
            pltpu.VMEM_SHARED((XROWS, H), F32),
            pltpu.VMEM_SHARED((ETOT,), I32),
            pltpu.VMEM((E2 // NS,), I32),
            pltpu.VMEM((E2 // NS,), I32),
            pltpu.VMEM((E2 // NS,), I32),
            pltpu.VMEM((K,), I32),
            pltpu.VMEM((K,), I32),
            pltpu.VMEM((K, H), F32),
            pltpu.VMEM((K, 8), F32),
            pltpu.SemaphoreType.DMA,
        ],
    )


def _dense_body(acc_lo, acc_hi, cnt_ref, xl_ref, xh_ref, ntf_ref, w_ref,
                r_ref, b_ref, *outs, last_t, relu, logsm, split_out):
    t = pl.program_id(1)

    @pl.when(t == 0)
    def _():
        xl = xl_ref[...]
        xh = xh_ref[...]
        m0 = ntf_ref[...] == 0.0
        r0 = (jnp.dot(xl, r_ref[0, :H], preferred_element_type=F32) +
              jnp.dot(xh, r_ref[0, H:], preferred_element_type=F32) +
              b_ref[0, :])
        r1 = (jnp.dot(xl, r_ref[1, :H], preferred_element_type=F32) +
              jnp.dot(xh, r_ref[1, H:], preferred_element_type=F32) +
              b_ref[1, :])
        y = jnp.where(m0, r0, r1)
        if split_out:
            outs[0][...] = y[:, :H]
            outs[1][...] = y[:, H:]
        else:
            outs[0][...] = y

    inv = 1.0 / jnp.maximum(cnt_ref[:, :1], 1.0)
    al = acc_lo[...] * inv
    ah = acc_hi[...] * inv
    y = (jnp.dot(al, w_ref[0, :H], preferred_element_type=F32) +
         jnp.dot(ah, w_ref[0, H:], preferred_element_type=F32))
    if split_out:
        outs[0][...] += y[:, :H]
        outs[1][...] += y[:, H:]
    else:
        outs[0][...] += y

    @pl.when(t == last_t)
    def _():
        if relu:
            for o in outs:
                o[...] = jnp.maximum(o[...], 0.0)
        if logsm:
            y2 = outs[0][...]
            m = jnp.max(y2, axis=-1, keepdims=True)
            e = jnp.exp(y2 - m)
            outs[0][...] = y2 - m - jnp.log(
                jnp.sum(e, axis=-1, keepdims=True))


def _make_dense(S, blk, relu, logsm, split_out):
    rb = S // blk
    body = functools.partial(_dense_body, last_t=ETY - 1, relu=relu,
                             logsm=logsm, split_out=split_out)
    amap = lambda i, t: (t * rb + i, 0)
    if split_out:
        out_specs = [pl.BlockSpec((blk, H), lambda i, t: (i, 0)),
                     pl.BlockSpec((blk, H), lambda i, t: (i, 0))]
        out_shape = [jax.ShapeDtypeStruct((S, H), F32),
                     jax.ShapeDtypeStruct((S, H), F32)]
    else:
        out_specs = pl.BlockSpec((blk, IN_C), lambda i, t: (i, 0))
        out_shape = jax.ShapeDtypeStruct((S, IN_C), F32)
    return pl.pallas_call(
        body,
        grid=(rb, ETY),
        in_specs=[
            pl.BlockSpec((blk, H), amap),
            pl.BlockSpec((blk, H), amap),
            pl.BlockSpec((blk, 8), amap),
            pl.BlockSpec((blk, H), lambda i, t: (i, 0)),
            pl.BlockSpec((blk, H), lambda i, t: (i, 0)),
            pl.BlockSpec((blk, 1), lambda i, t: (i, 0)),
            pl.BlockSpec((1, IN_C, IN_C), lambda i, t: (t, 0, 0)),
            pl.BlockSpec((NTY, IN_C, IN_C), lambda i, t: (0, 0, 0)),
            pl.BlockSpec((8, IN_C), lambda i, t: (0, 0)),
        ],
        out_specs=out_specs,
        out_shape=out_shape,
    )


_L1 = _make_l1()
_L2 = _make_l2()
_D1 = _make_dense(S1, 1000, relu=True, logsm=False, split_out=True)
_D2 = _make_dense(S2, 1000, relu=False, logsm=True, split_out=False)


def kernel(n_id, x0, emb1, edge_index1, e_id1, edge_index2, e_id2, edge_type,
           node_type, local_node_idx, rel_W1, root_W1, root_b1, rel_W2,
           root_W2, root_b2):
    tbl = jnp.concatenate([x0, emb1], axis=0)
    tbl_lo = tbl[:, :H]
    tbl_hi = tbl[:, H:]
    z64 = jnp.zeros((AP1 // NS, H), F32)
    z8 = jnp.zeros((AP1 // NS, 8), F32)
    z64b = jnp.zeros((AP2 // NS, H), F32)
    z8b = jnp.zeros((AP2 // NS, 8), F32)
    ones_in = jnp.ones((K, 8), F32)

    h_lo, h_hi, acc1a, acc1b, cnt1 = _L1(
        tbl_lo, tbl_hi, node_type, local_node_idx,
        edge_index1[0], edge_index1[1], e_id1, edge_type,
        z64, z8, ones_in)

    ntf1 = node_type[:S1].astype(F32)[:, None]
    b1p = jnp.zeros((8, IN_C), F32).at[:NTY].set(root_b1)

    x1_lo, x1_hi = _D1(acc1a, acc1b, cnt1, h_lo, h_hi, ntf1,
                       rel_W1, root_W1, b1p)

    acc2a, acc2b, cnt2 = _L2(
        x1_lo, x1_hi,
        edge_index2[0], edge_index2[1], e_id2, edge_type,
        z64b, z8b, ones_in)

    ntf2 = node_type[:S2].astype(F32)[:, None]
    b2p = jnp.zeros((8, IN_C), F32).at[:NTY].set(root_b2)

    return _D2(acc2a, acc2b, cnt2, x1_lo, x1_hi, ntf2, rel_W2, root_W2, b2p)


# speedup vs baseline: 31.6571x; 1.2042x over previous
import functools

import jax
import jax.numpy as jnp
from jax import lax
from jax.experimental import pallas as pl
from jax.experimental.pallas import tpu as pltpu
from jax.experimental.pallas import tpu_sc as plsc

F32 = jnp.float32
I32 = jnp.int32

IN_C = 128
NTY = 2
ETY = 4
N0 = 10000
S1 = 5000
S2 = 2000
E1 = 320000
E2 = 160000
ETOT = 480000
NX0 = 6000

NC = 2
NS = 16
L = 16
K = 80
H = 64
HROWS = 5120
XROWS = 2048
SCK = 2000
AP1 = 20480
AP2 = 8192


def _sc_layer1(tbl_lo, tbl_hi, nt_h, li_h, src_h, dst_h, eid_h, et_h,
               z64, z8, ones_in,
               h_lo, h_hi, acc_a, acc_b, cnt_o,
               acc_s, cnt_s, h_s,
               src_a, dst_a, eid_a, keyv, etv0, etv1, rowsv, rowsv1, onesv,
               ntv, liv, cidxv,
               sem, semE0, semE1, semR0, semR1):
    rpt = AP1 // NS
    ept = E1 // NS
    nch = ept // K
    hpt = HROWS // NS

    c = lax.axis_index("c")
    s = lax.axis_index("s")
    is0 = c == 0

    ab = s * rpt
    pltpu.sync_copy(z64, acc_s.at[pl.ds(ab, rpt)])

    @pl.when(is0)
    def _():
        pltpu.sync_copy(z8, cnt_s.at[pl.ds(ab, rpt)])

    pltpu.sync_copy(ones_in, onesv)

    for i in range(hpt // K):
        base = s * hpt + i * K
        pltpu.sync_copy(nt_h.at[pl.ds(base, K)], ntv)
        pltpu.sync_copy(li_h.at[pl.ds(base, K)], liv)
        for j in range(K // L):
            nt16 = ntv[pl.ds(j * L, L)]
            li16 = liv[pl.ds(j * L, L)]
            cidxv[pl.ds(j * L, L)] = jnp.where(nt16 == 0, li16, li16 + NX0)

        @pl.when(is0)
        def _():
            pltpu.async_copy(tbl_lo.at[cidxv], rowsv, sem).wait()
            pltpu.sync_copy(rowsv, h_lo.at[pl.ds(base, K)])

        @pl.when(~is0)
        def _():
            pltpu.async_copy(tbl_hi.at[cidxv], rowsv, sem).wait()
            pltpu.sync_copy(rowsv, h_hi.at[pl.ds(base, K)])

        pltpu.sync_copy(rowsv, h_s.at[pl.ds(base, K)])

    plsc.subcore_barrier()

    nchs = SCK // K

    def super_chunk(u, carry):
        ub = s * ept + u * SCK
        pltpu.sync_copy(src_h.at[pl.ds(ub, SCK)], src_a)
        pltpu.sync_copy(dst_h.at[pl.ds(ub, SCK)], dst_a)
        pltpu.sync_copy(eid_h.at[pl.ds(ub, SCK)], eid_a)
        pltpu.async_copy(et_h.at[eid_a.at[pl.ds(0, K)]], etv0, semE0)
        pltpu.async_copy(h_s.at[src_a.at[pl.ds(0, K)]], rowsv, semR0)

        def edge_chunk(i, carry2):
            base = i * K

            def do(cur, cursem, rcur, rcursem, nxt, nxtsem, rnxt, rnxtsem):
                @pl.when(i + 1 < nchs)
                def _():
                    pltpu.async_copy(
                        et_h.at[eid_a.at[pl.ds(base + K, K)]], nxt, nxtsem)
                    pltpu.async_copy(
                        h_s.at[src_a.at[pl.ds(base + K, K)]], rnxt, rnxtsem)

                pltpu.make_async_copy(et_h.at[pl.ds(0, K)], cur,
                                      cursem).wait()
                for j in range(K // L):
                    et16 = cur[pl.ds(j * L, L)]
                    d16 = dst_a[pl.ds(base + j * L, L)]
                    keyv[pl.ds(j * L, L)] = et16 * S1 + d16

                pltpu.make_async_copy(h_s.at[pl.ds(0, K)], rcur,
                                      rcursem).wait()
                pltpu.sync_copy(rcur, acc_s.at[keyv], add=True)

                @pl.when(is0)
                def _():
                    pltpu.sync_copy(onesv, cnt_s.at[keyv], add=True)

            @pl.when(i % 2 == 0)
            def _():
                do(etv0, semE0, rowsv, semR0, etv1, semE1, rowsv1, semR1)

            @pl.when(i % 2 == 1)
            def _():
                do(etv1, semE1, rowsv1, semR1, etv0, semE0, rowsv, semR0)

            return carry2

        lax.fori_loop(0, nchs, edge_chunk, 0)
        return carry

    lax.fori_loop(0, ept // SCK, super_chunk, 0)
    plsc.subcore_barrier()

    @pl.when(is0)
    def _():
        pltpu.sync_copy(acc_s.at[pl.ds(ab, rpt)], acc_a.at[pl.ds(ab, rpt)])
        pltpu.sync_copy(cnt_s.at[pl.ds(ab, rpt)], cnt_o.at[pl.ds(ab, rpt)])

    @pl.when(~is0)
    def _():
        pltpu.sync_copy(acc_s.at[pl.ds(ab, rpt)], acc_b.at[pl.ds(ab, rpt)])


def _sc_layer2(x_lo, x_hi, src_h, dst_h, eid_h, et_h, z64, z8, ones_in,
               acc_a, acc_b, cnt_o,
               acc_s, cnt_s, x_s, et_s,
               src_a, dst_a, eid_a, keyv, etv, etv1, rowsv, rowsv1, onesv,
               semE0, semE1, semR0, semR1):
    rpt = AP2 // NS
    ept = E2 // NS
    nch = ept // K
    xpt = XROWS // NS
    etpt = ETOT // NS

    c = lax.axis_index("c")
    s = lax.axis_index("s")
    is0 = c == 0

    ab = s * rpt
    pltpu.sync_copy(z64, acc_s.at[pl.ds(ab, rpt)])

    @pl.when(is0)
    def _():
        pltpu.sync_copy(z8, cnt_s.at[pl.ds(ab, rpt)])

    @pl.when(is0)
    def _():
        pltpu.sync_copy(x_lo.at[pl.ds(s * xpt, xpt)], x_s.at[pl.ds(s * xpt, xpt)])

    @pl.when(~is0)
    def _():
        pltpu.sync_copy(x_hi.at[pl.ds(s * xpt, xpt)], x_s.at[pl.ds(s * xpt, xpt)])

    pltpu.sync_copy(et_h.at[pl.ds(s * etpt, etpt)], et_s.at[pl.ds(s * etpt, etpt)])
    pltpu.sync_copy(ones_in, onesv)

    eb = s * ept
    pltpu.sync_copy(src_h.at[pl.ds(eb, ept)], src_a)
    pltpu.sync_copy(dst_h.at[pl.ds(eb, ept)], dst_a)
    pltpu.sync_copy(eid_h.at[pl.ds(eb, ept)], eid_a)
    plsc.subcore_barrier()

    pltpu.async_copy(et_s.at[eid_a.at[pl.ds(0, K)]], etv, semE0)
    pltpu.async_copy(x_s.at[src_a.at[pl.ds(0, K)]], rowsv, semR0)

    def edge_chunk(i, carry):
        base = i * K

        def do(cur, cursem, rcur, rcursem, nxt, nxtsem, rnxt, rnxtsem):
            @pl.when(i + 1 < nch)
            def _():
                pltpu.async_copy(
                    et_s.at[eid_a.at[pl.ds(base + K, K)]], nxt, nxtsem)
                pltpu.async_copy(
                    x_s.at[src_a.at[pl.ds(base + K, K)]], rnxt, rnxtsem)

            pltpu.make_async_copy(et_s.at[pl.ds(0, K)], cur, cursem).wait()
            for j in range(K // L):
                et16 = cur[pl.ds(j * L, L)]
                d16 = dst_a[pl.ds(base + j * L, L)]
                keyv[pl.ds(j * L, L)] = et16 * S2 + d16

            pltpu.make_async_copy(x_s.at[pl.ds(0, K)], rcur, rcursem).wait()
            pltpu.sync_copy(rcur, acc_s.at[keyv], add=True)

            @pl.when(is0)
            def _():
                pltpu.sync_copy(onesv, cnt_s.at[keyv], add=True)

        @pl.when(i % 2 == 0)
        def _():
            do(etv, semE0, rowsv, semR0, etv1, semE1, rowsv1, semR1)

        @pl.when(i % 2 == 1)
        def _():
            do(etv1, semE1, rowsv1, semR1, etv, semE0, rowsv, semR0)

        return carry

    lax.fori_loop(0, nch, edge_chunk, 0)
    plsc.subcore_barrier()

    @pl.when(is0)
    def _():
        pltpu.sync_copy(acc_s.at[pl.ds(ab, rpt)], acc_a.at[pl.ds(ab, rpt)])
        pltpu.sync_copy(cnt_s.at[pl.ds(ab, rpt)], cnt_o.at[pl.ds(ab, rpt)])

    @pl.when(~is0)
    def _():
        pltpu.sync_copy(acc_s.at[pl.ds(ab, rpt)], acc_b.at[pl.ds(ab, rpt)])


def _make_l1():
    A = AP1
    mesh = plsc.VectorSubcoreMesh(core_axis_name="c", subcore_axis_name="s")
    return pl.kernel(
        _sc_layer1,
        compiler_params=pltpu.CompilerParams(use_tc_tiling_on_sc=False),
        out_type=[
            jax.ShapeDtypeStruct((HROWS, H), F32),
            jax.ShapeDtypeStruct((HROWS, H), F32),
            jax.ShapeDtypeStruct((A, H), F32),
            jax.ShapeDtypeStruct((A, H), F32),
            jax.ShapeDtypeStruct((A, 8), F32),
        ],
        mesh=mesh,
        scratch_types=[
            pltpu.VMEM_SHARED((A, H), F32),
            pltpu.VMEM_SHARED((A, 8), F32),
            pltpu.VMEM_SHARED((HROWS, H), F32),
            pltpu.VMEM((SCK,), I32),
            pltpu.VMEM((SCK,), I32),
            pltpu.VMEM((SCK,), I32),
            pltpu.VMEM((K,), I32),
            pltpu.VMEM((K,), I32),
            pltpu.VMEM((K,), I32),
            pltpu.VMEM((K, H), F32),
            pltpu.VMEM((K, H), F32),
            pltpu.VMEM((K, 8), F32),
            pltpu.VMEM((K,), I32),
            pltpu.VMEM((K,), I32),
            pltpu.VMEM((K,), I32),
            pltpu.SemaphoreType.DMA,
            pltpu.SemaphoreType.DMA,
            pltpu.SemaphoreType.DMA,
            pltpu.SemaphoreType.DMA,
            pltpu.SemaphoreType.DMA,
        ],
    )


def _make_l2():
    A = AP2
    mesh = plsc.VectorSubcoreMesh(core_axis_name="c", subcore_axis_name="s")
    return pl.kernel(
        _sc_layer2,
        compiler_params=pltpu.CompilerParams(use_tc_tiling_on_sc=False),
        out_type=[
            jax.ShapeDtypeStruct((A, H), F32),
            jax.ShapeDtypeStruct((A, H), F32),
            jax.ShapeDtypeStruct((A, 8), F32),
        ],
        mesh=mesh,
        scratch_types=[
            pltpu.VMEM_SHARED((A, H), F32),
            pltpu.VMEM_SHARED((A, 8), F32),
            pltpu.VMEM_SHARED((XROWS, H), F32),
            pltpu.VMEM_SHARED((ETOT,), I32),
            pltpu.VMEM((E2 // NS,), I32),
            pltpu.VMEM((E2 // NS,), I32),
            pltpu.VMEM((E2 // NS,), I32),
            pltpu.VMEM((K,), I32),
            pltpu.VMEM((K,), I32),
            pltpu.VMEM((K,), I32),
            pltpu.VMEM((K, H), F32),
            pltpu.VMEM((K, H), F32),
            pltpu.VMEM((K, 8), F32),
            pltpu.SemaphoreType.DMA,
            pltpu.SemaphoreType.DMA,
            pltpu.SemaphoreType.DMA,
            pltpu.SemaphoreType.DMA,
        ],
    )


def _dense_body(acc_lo, acc_hi, cnt_ref, xl_ref, xh_ref, ntf_ref, w_ref,
                r_ref, b_ref, *outs, last_t, relu, logsm, split_out):
    t = pl.program_id(1)

    @pl.when(t == 0)
    def _():
        xl = xl_ref[...]
        xh = xh_ref[...]
        m0 = ntf_ref[...] == 0.0
        r0 = (jnp.dot(xl, r_ref[0, :H], preferred_element_type=F32) +
              jnp.dot(xh, r_ref[0, H:], preferred_element_type=F32) +
              b_ref[0, :])
        r1 = (jnp.dot(xl, r_ref[1, :H], preferred_element_type=F32) +
              jnp.dot(xh, r_ref[1, H:], preferred_element_type=F32) +
              b_ref[1, :])
        y = jnp.where(m0, r0, r1)
        if split_out:
            outs[0][...] = y[:, :H]
            outs[1][...] = y[:, H:]
        else:
            outs[0][...] = y

    inv = 1.0 / jnp.maximum(cnt_ref[:, :1], 1.0)
    al = acc_lo[...] * inv
    ah = acc_hi[...] * inv
    y = (jnp.dot(al, w_ref[0, :H], preferred_element_type=F32) +
         jnp.dot(ah, w_ref[0, H:], preferred_element_type=F32))
    if split_out:
        outs[0][...] += y[:, :H]
        outs[1][...] += y[:, H:]
    else:
        outs[0][...] += y

    @pl.when(t == last_t)
    def _():
        if relu:
            for o in outs:
                o[...] = jnp.maximum(o[...], 0.0)
        if logsm:
            y2 = outs[0][...]
            m = jnp.max(y2, axis=-1, keepdims=True)
            e = jnp.exp(y2 - m)
            outs[0][...] = y2 - m - jnp.log(
                jnp.sum(e, axis=-1, keepdims=True))


def _make_dense(S, blk, relu, logsm, split_out):
    rb = S // blk
    body = functools.partial(_dense_body, last_t=ETY - 1, relu=relu,
                             logsm=logsm, split_out=split_out)
    amap = lambda i, t: (t * rb + i, 0)
    if split_out:
        out_specs = [pl.BlockSpec((blk, H), lambda i, t: (i, 0)),
                     pl.BlockSpec((blk, H), lambda i, t: (i, 0))]
        out_shape = [jax.ShapeDtypeStruct((S, H), F32),
                     jax.ShapeDtypeStruct((S, H), F32)]
    else:
        out_specs = pl.BlockSpec((blk, IN_C), lambda i, t: (i, 0))
        out_shape = jax.ShapeDtypeStruct((S, IN_C), F32)
    return pl.pallas_call(
        body,
        grid=(rb, ETY),
        in_specs=[
            pl.BlockSpec((blk, H), amap),
            pl.BlockSpec((blk, H), amap),
            pl.BlockSpec((blk, 8), amap),
            pl.BlockSpec((blk, H), lambda i, t: (i, 0)),
            pl.BlockSpec((blk, H), lambda i, t: (i, 0)),
            pl.BlockSpec((blk, 1), lambda i, t: (i, 0)),
            pl.BlockSpec((1, IN_C, IN_C), lambda i, t: (t, 0, 0)),
            pl.BlockSpec((NTY, IN_C, IN_C), lambda i, t: (0, 0, 0)),
            pl.BlockSpec((8, IN_C), lambda i, t: (0, 0)),
        ],
        out_specs=out_specs,
        out_shape=out_shape,
    )


_L1 = _make_l1()
_L2 = _make_l2()
_D1 = _make_dense(S1, 1000, relu=True, logsm=False, split_out=True)
_D2 = _make_dense(S2, 1000, relu=False, logsm=True, split_out=False)


def kernel(n_id, x0, emb1, edge_index1, e_id1, edge_index2, e_id2, edge_type,
           node_type, local_node_idx, rel_W1, root_W1, root_b1, rel_W2,
           root_W2, root_b2):
    tbl = jnp.concatenate([x0, emb1], axis=0)
    tbl_lo = tbl[:, :H]
    tbl_hi = tbl[:, H:]
    z64 = jnp.zeros((AP1 // NS, H), F32)
    z8 = jnp.zeros((AP1 // NS, 8), F32)
    z64b = jnp.zeros((AP2 // NS, H), F32)
    z8b = jnp.zeros((AP2 // NS, 8), F32)
    ones_in = jnp.ones((K, 8), F32)

    h_lo, h_hi, acc1a, acc1b, cnt1 = _L1(
        tbl_lo, tbl_hi, node_type, local_node_idx,
        edge_index1[0], edge_index1[1], e_id1, edge_type,
        z64, z8, ones_in)

    ntf1 = node_type[:S1].astype(F32)[:, None]
    b1p = jnp.zeros((8, IN_C), F32).at[:NTY].set(root_b1)

    x1_lo, x1_hi = _D1(acc1a, acc1b, cnt1, h_lo, h_hi, ntf1,
                       rel_W1, root_W1, b1p)

    acc2a, acc2b, cnt2 = _L2(
        x1_lo, x1_hi,
        edge_index2[0], edge_index2[1], e_id2, edge_type,
        z64b, z8b, ones_in)

    ntf2 = node_type[:S2].astype(F32)[:, None]
    b2p = jnp.zeros((8, IN_C), F32).at[:NTY].set(root_b2)

    return _D2(acc2a, acc2b, cnt2, x1_lo, x1_hi, ntf2, rel_W2, root_W2, b2p)


# speedup vs baseline: 32.1838x; 1.0166x over previous
import functools

import jax
import jax.numpy as jnp
from jax import lax
from jax.experimental import pallas as pl
from jax.experimental.pallas import tpu as pltpu
from jax.experimental.pallas import tpu_sc as plsc

F32 = jnp.float32
I32 = jnp.int32

IN_C = 128
NTY = 2
ETY = 4
N0 = 10000
S1 = 5000
S2 = 2000
E1 = 320000
E2 = 160000
ETOT = 480000
NX0 = 6000

NC = 2
NS = 16
L = 16
K = 80
H = 64
HROWS = 5120
XROWS = 2048
SCK = 2000
AP1 = 20480
AP2 = 8192


def _sc_layer1(tbl_lo, tbl_hi, nt_h, li_h, src_h, dst_h, eid_h, et_h,
               zk64, zk8, ones_in,
               h_lo, h_hi, acc_a, acc_b, cnt_o,
               acc_s, cnt_s, h_s,
               src_a, dst_a, eid_a, keyv, etv0, etv1, rowsv, rowsv1, onesv,
               cidxv,
               sem, semE0, semE1, semR0, semR1):
    rpt = AP1 // NS
    ept = E1 // NS
    nch = ept // K
    hpt = HROWS // NS

    c = lax.axis_index("c")
    s = lax.axis_index("s")
    is0 = c == 0

    ab = s * rpt

    pltpu.sync_copy(zk64, rowsv)
    for r in range(rpt // K):
        pltpu.sync_copy(rowsv, acc_s.at[pl.ds(ab + r * K, K)])

    @pl.when(is0)
    def _():
        pltpu.sync_copy(zk8, onesv)
        for r in range(rpt // K):
            pltpu.sync_copy(onesv, cnt_s.at[pl.ds(ab + r * K, K)])

    pltpu.sync_copy(ones_in, onesv)

    nchA = hpt // K
    hb = s * hpt
    pltpu.sync_copy(nt_h.at[pl.ds(hb, hpt)], src_a.at[pl.ds(0, hpt)])
    pltpu.sync_copy(li_h.at[pl.ds(hb, hpt)], dst_a.at[pl.ds(0, hpt)])

    def cidx_for(i, buf):
        for j in range(K // L):
            nt16 = src_a[pl.ds(i * K + j * L, L)]
            li16 = dst_a[pl.ds(i * K + j * L, L)]
            buf[pl.ds(j * L, L)] = jnp.where(nt16 == 0, li16, li16 + NX0)

    ibufs = [cidxv, keyv]
    rbufs = [rowsv, rowsv1]
    rsems = [semR0, semR1]
    cidx_for(0, ibufs[0])

    @pl.when(is0)
    def _():
        pltpu.async_copy(tbl_lo.at[ibufs[0]], rbufs[0], rsems[0])

    @pl.when(~is0)
    def _():
        pltpu.async_copy(tbl_hi.at[ibufs[0]], rbufs[0], rsems[0])

    for i in range(nchA):
        p, q = i % 2, (i + 1) % 2
        if i + 1 < nchA:
            cidx_for(i + 1, ibufs[q])

            @pl.when(is0)
            def _():
                pltpu.async_copy(tbl_lo.at[ibufs[q]], rbufs[q], rsems[q])

            @pl.when(~is0)
            def _():
                pltpu.async_copy(tbl_hi.at[ibufs[q]], rbufs[q], rsems[q])

        pltpu.make_async_copy(tbl_lo.at[pl.ds(0, K)], rbufs[p],
                              rsems[p]).wait()
        pltpu.sync_copy(rbufs[p], h_s.at[pl.ds(hb + i * K, K)])

    @pl.when(is0)
    def _():
        pltpu.async_copy(h_s.at[pl.ds(hb, hpt)], h_lo.at[pl.ds(hb, hpt)], sem)

    @pl.when(~is0)
    def _():
        pltpu.async_copy(h_s.at[pl.ds(hb, hpt)], h_hi.at[pl.ds(hb, hpt)], sem)

    plsc.subcore_barrier()

    nchs = SCK // K

    def super_chunk(u, carry):
        ub = s * ept + u * SCK
        pltpu.sync_copy(src_h.at[pl.ds(ub, SCK)], src_a)
        pltpu.sync_copy(dst_h.at[pl.ds(ub, SCK)], dst_a)
        pltpu.sync_copy(eid_h.at[pl.ds(ub, SCK)], eid_a)
        pltpu.async_copy(et_h.at[eid_a.at[pl.ds(0, K)]], etv0, semE0)
        pltpu.async_copy(h_s.at[src_a.at[pl.ds(0, K)]], rowsv, semR0)

        def edge_chunk(i, carry2):
            base = i * K

            def do(cur, cursem, rcur, rcursem, nxt, nxtsem, rnxt, rnxtsem):
                @pl.when(i + 1 < nchs)
                def _():
                    pltpu.async_copy(
                        et_h.at[eid_a.at[pl.ds(base + K, K)]], nxt, nxtsem)
                    pltpu.async_copy(
                        h_s.at[src_a.at[pl.ds(base + K, K)]], rnxt, rnxtsem)

                pltpu.make_async_copy(et_h.at[pl.ds(0, K)], cur,
                                      cursem).wait()
                for j in range(K // L):
                    et16 = cur[pl.ds(j * L, L)]
                    d16 = dst_a[pl.ds(base + j * L, L)]
                    keyv[pl.ds(j * L, L)] = et16 * S1 + d16

                pltpu.make_async_copy(h_s.at[pl.ds(0, K)], rcur,
                                      rcursem).wait()
                pltpu.sync_copy(rcur, acc_s.at[keyv], add=True)

                @pl.when(is0)
                def _():
                    pltpu.sync_copy(onesv, cnt_s.at[keyv], add=True)

            @pl.when(i % 2 == 0)
            def _():
                do(etv0, semE0, rowsv, semR0, etv1, semE1, rowsv1, semR1)

            @pl.when(i % 2 == 1)
            def _():
                do(etv1, semE1, rowsv1, semR1, etv0, semE0, rowsv, semR0)

            return carry2

        lax.fori_loop(0, nchs, edge_chunk, 0)
        return carry

    lax.fori_loop(0, ept // SCK, super_chunk, 0)
    plsc.subcore_barrier()

    @pl.when(is0)
    def _():
        pltpu.make_async_copy(h_s.at[pl.ds(hb, hpt)],
                              h_lo.at[pl.ds(hb, hpt)], sem).wait()
        pltpu.sync_copy(acc_s.at[pl.ds(ab, rpt)], acc_a.at[pl.ds(ab, rpt)])
        pltpu.sync_copy(cnt_s.at[pl.ds(ab, rpt)], cnt_o.at[pl.ds(ab, rpt)])

    @pl.when(~is0)
    def _():
        pltpu.make_async_copy(h_s.at[pl.ds(hb, hpt)],
                              h_hi.at[pl.ds(hb, hpt)], sem).wait()
        pltpu.sync_copy(acc_s.at[pl.ds(ab, rpt)], acc_b.at[pl.ds(ab, rpt)])


def _sc_layer2(x_lo, x_hi, src_h, dst_h, eid_h, et_h, zk64, zk8, ones_in,
               acc_a, acc_b, cnt_o,
               acc_s, cnt_s, x_s, et_s,
               src_a, dst_a, eid_a, keyv, etv, etv1, rowsv, rowsv1, onesv,
               semE0, semE1, semR0, semR1):
    rpt = AP2 // NS
    ept = E2 // NS
    nch = ept // K
    xpt = XROWS // NS
    etpt = ETOT // NS

    c = lax.axis_index("c")
    s = lax.axis_index("s")
    is0 = c == 0

    ab = s * rpt
    pltpu.sync_copy(zk64, rowsv)
    for r in range(rpt // K):
        pltpu.sync_copy(rowsv, acc_s.at[pl.ds(ab + r * K, K)])
    if rpt % K:
        pltpu.sync_copy(rowsv.at[pl.ds(0, rpt % K)],
                        acc_s.at[pl.ds(ab + (rpt // K) * K, rpt % K)])

    @pl.when(is0)
    def _():
        pltpu.sync_copy(zk8, onesv)
        for r in range(rpt // K):
            pltpu.sync_copy(onesv, cnt_s.at[pl.ds(ab + r * K, K)])
        if rpt % K:
            pltpu.sync_copy(onesv.at[pl.ds(0, rpt % K)],
                            cnt_s.at[pl.ds(ab + (rpt // K) * K, rpt % K)])

    @pl.when(is0)
    def _():
        pltpu.sync_copy(x_lo.at[pl.ds(s * xpt, xpt)], x_s.at[pl.ds(s * xpt, xpt)])

    @pl.when(~is0)
    def _():
        pltpu.sync_copy(x_hi.at[pl.ds(s * xpt, xpt)], x_s.at[pl.ds(s * xpt, xpt)])

    pltpu.sync_copy(et_h.at[pl.ds(s * etpt, etpt)], et_s.at[pl.ds(s * etpt, etpt)])
    pltpu.sync_copy(ones_in, onesv)

    eb = s * ept
    pltpu.sync_copy(src_h.at[pl.ds(eb, ept)], src_a)
    pltpu.sync_copy(dst_h.at[pl.ds(eb, ept)], dst_a)
    pltpu.sync_copy(eid_h.at[pl.ds(eb, ept)], eid_a)
    plsc.subcore_barrier()

    pltpu.async_copy(et_s.at[eid_a.at[pl.ds(0, K)]], etv, semE0)
    pltpu.async_copy(x_s.at[src_a.at[pl.ds(0, K)]], rowsv, semR0)

    def edge_chunk(i, carry):
        base = i * K

        def do(cur, cursem, rcur, rcursem, nxt, nxtsem, rnxt, rnxtsem):
            @pl.when(i + 1 < nch)
            def _():
                pltpu.async_copy(
                    et_s.at[eid_a.at[pl.ds(base + K, K)]], nxt, nxtsem)
                pltpu.async_copy(
                    x_s.at[src_a.at[pl.ds(base + K, K)]], rnxt, rnxtsem)

            pltpu.make_async_copy(et_s.at[pl.ds(0, K)], cur, cursem).wait()
            for j in range(K // L):
                et16 = cur[pl.ds(j * L, L)]
                d16 = dst_a[pl.ds(base + j * L, L)]
                keyv[pl.ds(j * L, L)] = et16 * S2 + d16

            pltpu.make_async_copy(x_s.at[pl.ds(0, K)], rcur, rcursem).wait()
            pltpu.sync_copy(rcur, acc_s.at[keyv], add=True)

            @pl.when(is0)
            def _():
                pltpu.sync_copy(onesv, cnt_s.at[keyv], add=True)

        @pl.when(i % 2 == 0)
        def _():
            do(etv, semE0, rowsv, semR0, etv1, semE1, rowsv1, semR1)

        @pl.when(i % 2 == 1)
        def _():
            do(etv1, semE1, rowsv1, semR1, etv, semE0, rowsv, semR0)

        return carry

    lax.fori_loop(0, nch, edge_chunk, 0)
    plsc.subcore_barrier()

    @pl.when(is0)
    def _():
        pltpu.sync_copy(acc_s.at[pl.ds(ab, rpt)], acc_a.at[pl.ds(ab, rpt)])
        pltpu.sync_copy(cnt_s.at[pl.ds(ab, rpt)], cnt_o.at[pl.ds(ab, rpt)])

    @pl.when(~is0)
    def _():
        pltpu.sync_copy(acc_s.at[pl.ds(ab, rpt)], acc_b.at[pl.ds(ab, rpt)])


def _make_l1():
    A = AP1
    mesh = plsc.VectorSubcoreMesh(core_axis_name="c", subcore_axis_name="s")
    return pl.kernel(
        _sc_layer1,
        compiler_params=pltpu.CompilerParams(use_tc_tiling_on_sc=False),
        out_type=[
            jax.ShapeDtypeStruct((HROWS, H), F32),
            jax.ShapeDtypeStruct((HROWS, H), F32),
            jax.ShapeDtypeStruct((A, H), F32),
            jax.ShapeDtypeStruct((A, H), F32),
            jax.ShapeDtypeStruct((A, 8), F32),
        ],
        mesh=mesh,
        scratch_types=[
            pltpu.VMEM_SHARED((A, H), F32),
            pltpu.VMEM_SHARED((A, 8), F32),
            pltpu.VMEM_SHARED((HROWS, H), F32),
            pltpu.VMEM((SCK,), I32),
            pltpu.VMEM((SCK,), I32),
            pltpu.VMEM((SCK,), I32),
            pltpu.VMEM((K,), I32),
            pltpu.VMEM((K,), I32),
            pltpu.VMEM((K,), I32),
            pltpu.VMEM((K, H), F32),
            pltpu.VMEM((K, H), F32),
            pltpu.VMEM((K, 8), F32),
            pltpu.VMEM((K,), I32),
            pltpu.SemaphoreType.DMA,
            pltpu.SemaphoreType.DMA,
            pltpu.SemaphoreType.DMA,
            pltpu.SemaphoreType.DMA,
            pltpu.SemaphoreType.DMA,
        ],
    )


def _make_l2():
    A = AP2
    mesh = plsc.VectorSubcoreMesh(core_axis_name="c", subcore_axis_name="s")
    return pl.kernel(
        _sc_layer2,
        compiler_params=pltpu.CompilerParams(use_tc_tiling_on_sc=False),
        out_type=[
            jax.ShapeDtypeStruct((A, H), F32),
            jax.ShapeDtypeStruct((A, H), F32),
            jax.ShapeDtypeStruct((A, 8), F32),
        ],
        mesh=mesh,
        scratch_types=[
            pltpu.VMEM_SHARED((A, H), F32),
            pltpu.VMEM_SHARED((A, 8), F32),
            pltpu.VMEM_SHARED((XROWS, H), F32),
            pltpu.VMEM_SHARED((ETOT,), I32),
            pltpu.VMEM((E2 // NS,), I32),
            pltpu.VMEM((E2 // NS,), I32),
            pltpu.VMEM((E2 // NS,), I32),
            pltpu.VMEM((K,), I32),
            pltpu.VMEM((K,), I32),
            pltpu.VMEM((K,), I32),
            pltpu.VMEM((K, H), F32),
            pltpu.VMEM((K, H), F32),
            pltpu.VMEM((K, 8), F32),
            pltpu.SemaphoreType.DMA,
            pltpu.SemaphoreType.DMA,
            pltpu.SemaphoreType.DMA,
            pltpu.SemaphoreType.DMA,
        ],
    )


def _dense_body(acc_lo, acc_hi, cnt_ref, xl_ref, xh_ref, ntf_ref, w_ref,
                r_ref, b_ref, *outs, last_t, relu, logsm, split_out):
    t = pl.program_id(1)

    @pl.when(t == 0)
    def _():
        xl = xl_ref[...]
        xh = xh_ref[...]
        m0 = ntf_ref[...] == 0.0
        r0 = (jnp.dot(xl, r_ref[0, :H], preferred_element_type=F32) +
              jnp.dot(xh, r_ref[0, H:], preferred_element_type=F32) +
              b_ref[0, :])
        r1 = (jnp.dot(xl, r_ref[1, :H], preferred_element_type=F32) +
              jnp.dot(xh, r_ref[1, H:], preferred_element_type=F32) +
              b_ref[1, :])
        y = jnp.where(m0, r0, r1)
        if split_out:
            outs[0][...] = y[:, :H]
            outs[1][...] = y[:, H:]
        else:
            outs[0][...] = y

    inv = 1.0 / jnp.maximum(cnt_ref[:, :1], 1.0)
    al = acc_lo[...] * inv
    ah = acc_hi[...] * inv
    y = (jnp.dot(al, w_ref[0, :H], preferred_element_type=F32) +
         jnp.dot(ah, w_ref[0, H:], preferred_element_type=F32))
    if split_out:
        outs[0][...] += y[:, :H]
        outs[1][...] += y[:, H:]
    else:
        outs[0][...] += y

    @pl.when(t == last_t)
    def _():
        if relu:
            for o in outs:
                o[...] = jnp.maximum(o[...], 0.0)
        if logsm:
            y2 = outs[0][...]
            m = jnp.max(y2, axis=-1, keepdims=True)
            e = jnp.exp(y2 - m)
            outs[0][...] = y2 - m - jnp.log(
                jnp.sum(e, axis=-1, keepdims=True))


def _make_dense(S, blk, relu, logsm, split_out):
    rb = S // blk
    body = functools.partial(_dense_body, last_t=ETY - 1, relu=relu,
                             logsm=logsm, split_out=split_out)
    amap = lambda i, t: (t * rb + i, 0)
    if split_out:
        out_specs = [pl.BlockSpec((blk, H), lambda i, t: (i, 0)),
                     pl.BlockSpec((blk, H), lambda i, t: (i, 0))]
        out_shape = [jax.ShapeDtypeStruct((S, H), F32),
                     jax.ShapeDtypeStruct((S, H), F32)]
    else:
        out_specs = pl.BlockSpec((blk, IN_C), lambda i, t: (i, 0))
        out_shape = jax.ShapeDtypeStruct((S, IN_C), F32)
    return pl.pallas_call(
        body,
        grid=(rb, ETY),
        in_specs=[
            pl.BlockSpec((blk, H), amap),
            pl.BlockSpec((blk, H), amap),
            pl.BlockSpec((blk, 8), amap),
            pl.BlockSpec((blk, H), lambda i, t: (i, 0)),
            pl.BlockSpec((blk, H), lambda i, t: (i, 0)),
            pl.BlockSpec((blk, 1), lambda i, t: (i, 0)),
            pl.BlockSpec((1, IN_C, IN_C), lambda i, t: (t, 0, 0)),
            pl.BlockSpec((NTY, IN_C, IN_C), lambda i, t: (0, 0, 0)),
            pl.BlockSpec((8, IN_C), lambda i, t: (0, 0)),
        ],
        out_specs=out_specs,
        out_shape=out_shape,
    )


_L1 = _make_l1()
_L2 = _make_l2()
_D1 = _make_dense(S1, 1000, relu=True, logsm=False, split_out=True)
_D2 = _make_dense(S2, 1000, relu=False, logsm=True, split_out=False)


def kernel(n_id, x0, emb1, edge_index1, e_id1, edge_index2, e_id2, edge_type,
           node_type, local_node_idx, rel_W1, root_W1, root_b1, rel_W2,
           root_W2, root_b2):
    tbl = jnp.concatenate([x0, emb1], axis=0)
    tbl_lo = tbl[:, :H]
    tbl_hi = tbl[:, H:]
    zk64 = jnp.zeros((K, H), F32)
    zk8 = jnp.zeros((K, 8), F32)
    ones_in = jnp.ones((K, 8), F32)

    h_lo, h_hi, acc1a, acc1b, cnt1 = _L1(
        tbl_lo, tbl_hi, node_type, local_node_idx,
        edge_index1[0], edge_index1[1], e_id1, edge_type,
        zk64, zk8, ones_in)

    ntf1 = node_type[:S1].astype(F32)[:, None]
    b1p = jnp.zeros((8, IN_C), F32).at[:NTY].set(root_b1)

    x1_lo, x1_hi = _D1(acc1a, acc1b, cnt1, h_lo, h_hi, ntf1,
                       rel_W1, root_W1, b1p)

    acc2a, acc2b, cnt2 = _L2(
        x1_lo, x1_hi,
        edge_index2[0], edge_index2[1], e_id2, edge_type,
        zk64, zk8, ones_in)

    ntf2 = node_type[:S2].astype(F32)[:, None]
    b2p = jnp.zeros((8, IN_C), F32).at[:NTY].set(root_b2)

    return _D2(acc2a, acc2b, cnt2, x1_lo, x1_hi, ntf2, rel_W2, root_W2, b2p)
